# Initial kernel scaffold; baseline (speedup 1.0000x reference)
import functools
import jax
import jax.numpy as jnp
from jax import lax
from jax.experimental import pallas as pl
from jax.experimental.pallas import tpu as pltpu

B_K = 128
C1_K = 512


def _cdiv(a, b):
    return (a + b - 1) // b


def _mm_body(x_ref, w_ref, b_ref, o_ref, *, act):
    k = pl.program_id(1)

    @pl.when(k == 0)
    def _():
        o_ref[...] = jnp.zeros_like(o_ref)

    o_ref[...] += jnp.dot(x_ref[...], w_ref[...],
                          preferred_element_type=jnp.float32)

    @pl.when(k == pl.num_programs(1) - 1)
    def _():
        r = o_ref[...] + b_ref[...]
        if act == 1:
            r = jnp.maximum(r, 0.0)
        elif act == 2:
            r = jnp.where(r > 0, r, jnp.exp(jnp.minimum(r, 0.0)) - 1.0)
        o_ref[...] = r


def _mm(x, w, b, act=0, bm=None, bk=None):
    m, kk = x.shape
    _, n = w.shape
    if bm is None:
        bm = min(m, 1024)
    if bk is None:
        bk = kk if kk <= 2048 else 2048
    grid = (_cdiv(m, bm), _cdiv(kk, bk))
    return pl.pallas_call(
        functools.partial(_mm_body, act=act),
        grid=grid,
        in_specs=[
            pl.BlockSpec((bm, bk), lambda i, k: (i, k)),
            pl.BlockSpec((bk, n), lambda i, k: (k, 0)),
            pl.BlockSpec((1, n), lambda i, k: (0, 0)),
        ],
        out_specs=pl.BlockSpec((bm, n), lambda i, k: (i, 0)),
        out_shape=jax.ShapeDtypeStruct((m, n), jnp.float32),
        compiler_params=pltpu.CompilerParams(
            dimension_semantics=("parallel", "arbitrary")),
    )(x, w, b.reshape(1, -1))


def _bn_body(x_ref, o_ref):
    xv = x_ref[...]
    mu = jnp.mean(xv, axis=0, keepdims=True)
    var = jnp.mean((xv - mu) ** 2, axis=0, keepdims=True)
    o_ref[...] = (xv - mu) * lax.rsqrt(var + 1e-5)


def _bn_pl(x):
    return pl.pallas_call(
        _bn_body,
        out_shape=jax.ShapeDtypeStruct(x.shape, jnp.float32),
    )(x)


def _gat_jax(h_aug, src, dst, n):
    h = h_aug[:, :16]
    es = h_aug[:, 16]
    ed = h_aug[:, 17]
    sl = jnp.arange(n)
    s = jnp.concatenate([src, sl])
    d = jnp.concatenate([dst, sl])
    e = jax.nn.leaky_relu(es[s] + ed[d], 0.2)
    ex = jnp.exp(e)
    den = jax.ops.segment_sum(ex, d, num_segments=n)
    num = jax.ops.segment_sum(ex[:, None] * h[s], d, num_segments=n)
    return jnp.maximum(num / den[:, None], 0.0)


def kernel(drug_x, cell_x, W1, b1, W2, b2, W3, b3, Wd, bd, gW0, gas0, gad0,
           gW1, gas1, gad1, Wc1, bc1, Wc2, bc2, Wr1, br1, Wr2, br2, Wr3, br3,
           drug_edge_index, drug_batch, cell_edge_index, cluster0, cluster1):
    n_drug = drug_x.shape[0]
    n_cell = cell_x.shape[0]
    b = B_K
    genes = cluster0.shape[0]
    c0 = cluster1.shape[0]
    c1 = C1_K

    src, dst = drug_edge_index[0], drug_edge_index[1]

    h = drug_x
    reps = []
    for (W, bb) in ((W1, b1), (W2, b2), (W3, b3)):
        agg = jax.ops.segment_sum(h[src], dst, num_segments=n_drug)
        h = _mm(h + agg, W, bb, act=1)
        reps.append(h)
    node_rep = jnp.concatenate(reps, axis=-1)
    x_drug = jax.ops.segment_max(node_rep, drug_batch, num_segments=b)
    x_drug = _mm(x_drug, Wd, bd, act=1)

    csrc, cdst = cell_edge_index[0], cell_edge_index[1]
    va_s0 = gW0 @ gas0
    va_d0 = gW0 @ gad0
    W0_aug = jnp.concatenate([gW0, va_s0[:, None], va_d0[:, None]], axis=1)
    h_aug0 = _mm(cell_x, W0_aug, jnp.zeros((18,), jnp.float32))
    h1 = _gat_jax(h_aug0, csrc, cdst, n_cell)

    node = jnp.arange(n_cell)
    clus = cluster0[node % genes] + (node // genes) * c0
    n1 = b * c0
    x = jax.ops.segment_max(h1, clus, num_segments=n1)
    csrc2, cdst2 = clus[csrc], clus[cdst]
    x = _bn_pl(x)

    va_s1 = gW1 @ gas1
    va_d1 = gW1 @ gad1
    W1_aug = jnp.concatenate([gW1, va_s1[:, None], va_d1[:, None]], axis=1)
    h_aug1 = _mm(x, W1_aug, jnp.zeros((18,), jnp.float32))
    h2 = _gat_jax(h_aug1, csrc2, cdst2, n1)

    node = jnp.arange(n1)
    clus1 = cluster1[node % c0] + (node // c0) * c1
    n2 = b * c1
    x = jax.ops.segment_max(h2, clus1, num_segments=n2)
    x = _bn_pl(x)

    x_cell = x.reshape(b, c1 * 16)
    x_cell = _mm(x_cell, Wc1, bc1, act=1, bm=b)
    x_cell = _mm(x_cell, Wc2, bc2, act=1, bm=b)

    z = jnp.concatenate([x_drug, x_cell], axis=-1)
    z = _mm(z, Wr1, br1, act=2, bm=b)
    z = _mm(z, Wr2, br2, act=2, bm=b)
    w3p = jnp.concatenate([Wr3, jnp.zeros((Wr3.shape[0], 127), jnp.float32)],
                          axis=1)
    b3p = jnp.concatenate([br3, jnp.zeros((127,), jnp.float32)])
    out = _mm(z, w3p, b3p, act=0, bm=b)
    return out[:, :1]


# speedup vs baseline: 1.5205x; 1.5205x over previous
import functools
import jax
import jax.numpy as jnp
from jax import lax
from jax.experimental import pallas as pl
from jax.experimental.pallas import tpu as pltpu

B_K = 128
C1_K = 512


def _cdiv(a, b):
    return (a + b - 1) // b


def _mm_body(x_ref, w_ref, b_ref, o_ref, *, act):
    k = pl.program_id(1)

    @pl.when(k == 0)
    def _():
        o_ref[...] = jnp.zeros_like(o_ref)

    o_ref[...] += jnp.dot(x_ref[...], w_ref[...],
                          preferred_element_type=jnp.float32)

    @pl.when(k == pl.num_programs(1) - 1)
    def _():
        r = o_ref[...] + b_ref[...]
        if act == 1:
            r = jnp.maximum(r, 0.0)
        elif act == 2:
            r = jnp.where(r > 0, r, jnp.exp(jnp.minimum(r, 0.0)) - 1.0)
        o_ref[...] = r


def _mm(x, w, b, act=0, bm=None, bk=None):
    m, kk = x.shape
    _, n = w.shape
    if bm is None:
        bm = min(m, 1024)
    if bk is None:
        bk = kk if kk <= 2048 else 2048
    grid = (_cdiv(m, bm), _cdiv(kk, bk))
    return pl.pallas_call(
        functools.partial(_mm_body, act=act),
        grid=grid,
        in_specs=[
            pl.BlockSpec((bm, bk), lambda i, k: (i, k)),
            pl.BlockSpec((bk, n), lambda i, k: (k, 0)),
            pl.BlockSpec((1, n), lambda i, k: (0, 0)),
        ],
        out_specs=pl.BlockSpec((bm, n), lambda i, k: (i, 0)),
        out_shape=jax.ShapeDtypeStruct((m, n), jnp.float32),
        compiler_params=pltpu.CompilerParams(
            dimension_semantics=("parallel", "arbitrary")),
    )(x, w, b.reshape(1, -1))


def _bn_body(x_ref, a_ref, o_ref):
    xv = x_ref[...]
    a = a_ref[...]
    m = jnp.mean(xv, axis=0, keepdims=True)
    m2 = jnp.mean(xv * xv, axis=0, keepdims=True)
    mu = jnp.dot(m, a, preferred_element_type=jnp.float32)
    mu2 = jnp.dot(m2, a, preferred_element_type=jnp.float32)
    var = mu2 - mu * mu
    o_ref[...] = (xv - mu) * lax.rsqrt(var + 1e-5)


def _bn_pl(x):
    n, f = x.shape
    g = 128 // f
    xr = x.reshape(n // g, 128)
    lane = jnp.arange(128)
    a_mat = (lane[:, None] % f == lane[None, :] % f).astype(jnp.float32) / g
    out = pl.pallas_call(
        _bn_body,
        out_shape=jax.ShapeDtypeStruct(xr.shape, jnp.float32),
    )(xr, a_mat)
    return out.reshape(n, f)


def _gat_jax(h_aug, src, dst, n):
    h = h_aug[:, :16]
    es = h_aug[:, 16]
    ed = h_aug[:, 17]
    sl = jnp.arange(n)
    s = jnp.concatenate([src, sl])
    d = jnp.concatenate([dst, sl])
    e = jax.nn.leaky_relu(es[s] + ed[d], 0.2)
    ex = jnp.exp(e)
    den = jax.ops.segment_sum(ex, d, num_segments=n)
    num = jax.ops.segment_sum(ex[:, None] * h[s], d, num_segments=n)
    return jnp.maximum(num / den[:, None], 0.0)


def kernel(drug_x, cell_x, W1, b1, W2, b2, W3, b3, Wd, bd, gW0, gas0, gad0,
           gW1, gas1, gad1, Wc1, bc1, Wc2, bc2, Wr1, br1, Wr2, br2, Wr3, br3,
           drug_edge_index, drug_batch, cell_edge_index, cluster0, cluster1):
    n_drug = drug_x.shape[0]
    n_cell = cell_x.shape[0]
    b = B_K
    genes = cluster0.shape[0]
    c0 = cluster1.shape[0]
    c1 = C1_K

    src, dst = drug_edge_index[0], drug_edge_index[1]

    h = drug_x
    reps = []
    for (W, bb) in ((W1, b1), (W2, b2), (W3, b3)):
        agg = jax.ops.segment_sum(h[src], dst, num_segments=n_drug)
        h = _mm(h + agg, W, bb, act=1)
        reps.append(h)
    node_rep = jnp.concatenate(reps, axis=-1)
    x_drug = jax.ops.segment_max(node_rep, drug_batch, num_segments=b)
    x_drug = _mm(x_drug, Wd, bd, act=1)

    csrc, cdst = cell_edge_index[0], cell_edge_index[1]
    va_s0 = gW0 @ gas0
    va_d0 = gW0 @ gad0
    W0_aug = jnp.concatenate([gW0, va_s0[:, None], va_d0[:, None]], axis=1)
    h_aug0 = _mm(cell_x, W0_aug, jnp.zeros((18,), jnp.float32))
    h1 = _gat_jax(h_aug0, csrc, cdst, n_cell)

    node = jnp.arange(n_cell)
    clus = cluster0[node % genes] + (node // genes) * c0
    n1 = b * c0
    x = jax.ops.segment_max(h1, clus, num_segments=n1)
    csrc2, cdst2 = clus[csrc], clus[cdst]
    x = _bn_pl(x)

    va_s1 = gW1 @ gas1
    va_d1 = gW1 @ gad1
    W1_aug = jnp.concatenate([gW1, va_s1[:, None], va_d1[:, None]], axis=1)
    h_aug1 = _mm(x, W1_aug, jnp.zeros((18,), jnp.float32))
    h2 = _gat_jax(h_aug1, csrc2, cdst2, n1)

    node = jnp.arange(n1)
    clus1 = cluster1[node % c0] + (node // c0) * c1
    n2 = b * c1
    x = jax.ops.segment_max(h2, clus1, num_segments=n2)
    x = _bn_pl(x)

    x_cell = x.reshape(b, c1 * 16)
    x_cell = _mm(x_cell, Wc1, bc1, act=1, bm=b)
    x_cell = _mm(x_cell, Wc2, bc2, act=1, bm=b)

    z = jnp.concatenate([x_drug, x_cell], axis=-1)
    z = _mm(z, Wr1, br1, act=2, bm=b)
    z = _mm(z, Wr2, br2, act=2, bm=b)
    w3p = jnp.concatenate([Wr3, jnp.zeros((Wr3.shape[0], 127), jnp.float32)],
                          axis=1)
    b3p = jnp.concatenate([br3, jnp.zeros((127,), jnp.float32)])
    out = _mm(z, w3p, b3p, act=0, bm=b)
    return out[:, :1]


# speedup vs baseline: 1.5579x; 1.0246x over previous
import functools
import jax
import jax.numpy as jnp
from jax import lax
from jax.experimental import pallas as pl
from jax.experimental.pallas import tpu as pltpu
from jax.experimental.pallas import tpu_sc as plsc

_NC = 2
_NS = 16

B_K = 128
C1_K = 512


def _cdiv(a, b):
    return (a + b - 1) // b


def _mm_body(x_ref, w_ref, b_ref, o_ref, *, act):
    k = pl.program_id(1)

    @pl.when(k == 0)
    def _():
        o_ref[...] = jnp.zeros_like(o_ref)

    o_ref[...] += jnp.dot(x_ref[...], w_ref[...],
                          preferred_element_type=jnp.float32)

    @pl.when(k == pl.num_programs(1) - 1)
    def _():
        r = o_ref[...] + b_ref[...]
        if act == 1:
            r = jnp.maximum(r, 0.0)
        elif act == 2:
            r = jnp.where(r > 0, r, jnp.exp(jnp.minimum(r, 0.0)) - 1.0)
        o_ref[...] = r


def _mm(x, w, b, act=0, bm=None, bk=None):
    m, kk = x.shape
    _, n = w.shape
    if bm is None:
        bm = min(m, 1024)
    if bk is None:
        bk = kk if kk <= 2048 else 2048
    grid = (_cdiv(m, bm), _cdiv(kk, bk))
    return pl.pallas_call(
        functools.partial(_mm_body, act=act),
        grid=grid,
        in_specs=[
            pl.BlockSpec((bm, bk), lambda i, k: (i, k)),
            pl.BlockSpec((bk, n), lambda i, k: (k, 0)),
            pl.BlockSpec((1, n), lambda i, k: (0, 0)),
        ],
        out_specs=pl.BlockSpec((bm, n), lambda i, k: (i, 0)),
        out_shape=jax.ShapeDtypeStruct((m, n), jnp.float32),
        compiler_params=pltpu.CompilerParams(
            dimension_semantics=("parallel", "arbitrary")),
    )(x, w, b.reshape(1, -1))


def _bn_body(x_ref, a_ref, o_ref):
    xv = x_ref[...]
    a = a_ref[...]
    m = jnp.mean(xv, axis=0, keepdims=True)
    m2 = jnp.mean(xv * xv, axis=0, keepdims=True)
    mu = jnp.dot(m, a, preferred_element_type=jnp.float32)
    mu2 = jnp.dot(m2, a, preferred_element_type=jnp.float32)
    var = mu2 - mu * mu
    o_ref[...] = (xv - mu) * lax.rsqrt(var + 1e-5)


def _bn_pl(x):
    n, f = x.shape
    g = 128 // f
    xr = x.reshape(n // g, 128)
    lane = jnp.arange(128)
    a_mat = (lane[:, None] % f == lane[None, :] % f).astype(jnp.float32) / g
    out = pl.pallas_call(
        _bn_body,
        out_shape=jax.ShapeDtypeStruct(xr.shape, jnp.float32),
    )(xr, a_mat)
    return out.reshape(n, f)


def _sc_segsum(h, src, dst, k_chunk=80):
    n, d = h.shape
    e = src.shape[0]
    nw = _NC * _NS
    e_per = e // nw
    rowb = 80
    nb = _cdiv(n, rowb)
    assert e % nw == 0 and e_per % k_chunk == 0 and n % rowb == 0
    n_chunks = e_per // k_chunk
    mesh = plsc.VectorSubcoreMesh(core_axis_name="c", subcore_axis_name="s")

    @functools.partial(
        pl.kernel,
        out_type=jax.ShapeDtypeStruct((_NC, n, d), jnp.float32),
        mesh=mesh,
        scratch_types=[
            pltpu.VMEM_SHARED((n, d), jnp.float32),
            pltpu.VMEM((k_chunk,), jnp.int32),
            pltpu.VMEM((k_chunk,), jnp.int32),
            pltpu.VMEM((k_chunk, d), jnp.float32),
            pltpu.SemaphoreType.DMA,
        ],
    )
    def k(h_hbm, src_hbm, dst_hbm, out_hbm, acc, sidx, didx, rows, sem):
        cid = lax.axis_index("c")
        sid = lax.axis_index("s")
        for jj in range(_cdiv(nb, _NS)):
            j = jj * _NS + sid

            @pl.when(j < nb)
            def _():
                pltpu.sync_copy(h_hbm.at[pl.ds(j * rowb, rowb)],
                                acc.at[pl.ds(j * rowb, rowb)])
        plsc.subcore_barrier()
        wid = cid * _NS + sid

        def body(j, carry):
            base = wid * e_per + j * k_chunk
            pltpu.sync_copy(src_hbm.at[pl.ds(base, k_chunk)], sidx)
            pltpu.sync_copy(dst_hbm.at[pl.ds(base, k_chunk)], didx)
            pltpu.async_copy(h_hbm.at[sidx], rows, sem).wait()
            pltpu.sync_copy(rows, acc.at[didx], add=True)
            return carry

        lax.fori_loop(0, n_chunks, body, 0)
        plsc.subcore_barrier()
        for jj in range(_cdiv(nb, _NS)):
            j = jj * _NS + sid

            @pl.when(j < nb)
            def _():
                pltpu.sync_copy(acc.at[pl.ds(j * rowb, rowb)],
                                out_hbm.at[cid, pl.ds(j * rowb, rowb)])

    return k(h, src, dst)


def _mm_comb_body(p0_ref, p1_ref, h_ref, w_ref, b_ref, o_ref):
    x = p0_ref[0] + p1_ref[0] - h_ref[...]
    r = jnp.dot(x, w_ref[...], preferred_element_type=jnp.float32)
    o_ref[...] = jnp.maximum(r + b_ref[...], 0.0)


def _mm_comb(p, h, w, b, bm=1024):
    n, d = h.shape
    _, nn = w.shape
    grid = (_cdiv(n, bm),)
    return pl.pallas_call(
        _mm_comb_body,
        grid=grid,
        in_specs=[
            pl.BlockSpec((1, bm, d), lambda i: (0, i, 0)),
            pl.BlockSpec((1, bm, d), lambda i: (1, i, 0)),
            pl.BlockSpec((bm, d), lambda i: (i, 0)),
            pl.BlockSpec((d, nn), lambda i: (0, 0)),
            pl.BlockSpec((1, nn), lambda i: (0, 0)),
        ],
        out_specs=pl.BlockSpec((bm, nn), lambda i: (i, 0)),
        out_shape=jax.ShapeDtypeStruct((n, nn), jnp.float32),
        compiler_params=pltpu.CompilerParams(
            dimension_semantics=("parallel",)),
    )(p, p, h, w, b.reshape(1, -1))


def _gat_jax(h_aug, src, dst, n):
    h = h_aug[:, :16]
    es = h_aug[:, 16]
    ed = h_aug[:, 17]
    sl = jnp.arange(n)
    s = jnp.concatenate([src, sl])
    d = jnp.concatenate([dst, sl])
    e = jax.nn.leaky_relu(es[s] + ed[d], 0.2)
    ex = jnp.exp(e)
    den = jax.ops.segment_sum(ex, d, num_segments=n)
    num = jax.ops.segment_sum(ex[:, None] * h[s], d, num_segments=n)
    return jnp.maximum(num / den[:, None], 0.0)


def kernel(drug_x, cell_x, W1, b1, W2, b2, W3, b3, Wd, bd, gW0, gas0, gad0,
           gW1, gas1, gad1, Wc1, bc1, Wc2, bc2, Wr1, br1, Wr2, br2, Wr3, br3,
           drug_edge_index, drug_batch, cell_edge_index, cluster0, cluster1):
    n_drug = drug_x.shape[0]
    n_cell = cell_x.shape[0]
    b = B_K
    genes = cluster0.shape[0]
    c0 = cluster1.shape[0]
    c1 = C1_K

    src, dst = drug_edge_index[0], drug_edge_index[1]

    h = drug_x
    src32 = src.astype(jnp.int32)
    dst32 = dst.astype(jnp.int32)
    reps = []
    for (W, bb) in ((W1, b1), (W2, b2), (W3, b3)):
        p = _sc_segsum(h, src32, dst32)
        h = _mm_comb(p, h, W, bb)
        reps.append(h)
    node_rep = jnp.concatenate(reps, axis=-1)
    x_drug = jax.ops.segment_max(node_rep, drug_batch, num_segments=b)
    x_drug = _mm(x_drug, Wd, bd, act=1)

    csrc, cdst = cell_edge_index[0], cell_edge_index[1]
    va_s0 = gW0 @ gas0
    va_d0 = gW0 @ gad0
    W0_aug = jnp.concatenate([gW0, va_s0[:, None], va_d0[:, None]], axis=1)
    h_aug0 = _mm(cell_x, W0_aug, jnp.zeros((18,), jnp.float32))
    h1 = _gat_jax(h_aug0, csrc, cdst, n_cell)

    node = jnp.arange(n_cell)
    clus = cluster0[node % genes] + (node // genes) * c0
    n1 = b * c0
    x = jax.ops.segment_max(h1, clus, num_segments=n1)
    csrc2, cdst2 = clus[csrc], clus[cdst]
    x = _bn_pl(x)

    va_s1 = gW1 @ gas1
    va_d1 = gW1 @ gad1
    W1_aug = jnp.concatenate([gW1, va_s1[:, None], va_d1[:, None]], axis=1)
    h_aug1 = _mm(x, W1_aug, jnp.zeros((18,), jnp.float32))
    h2 = _gat_jax(h_aug1, csrc2, cdst2, n1)

    node = jnp.arange(n1)
    clus1 = cluster1[node % c0] + (node // c0) * c1
    n2 = b * c1
    x = jax.ops.segment_max(h2, clus1, num_segments=n2)
    x = _bn_pl(x)

    x_cell = x.reshape(b, c1 * 16)
    x_cell = _mm(x_cell, Wc1, bc1, act=1, bm=b)
    x_cell = _mm(x_cell, Wc2, bc2, act=1, bm=b)

    z = jnp.concatenate([x_drug, x_cell], axis=-1)
    z = _mm(z, Wr1, br1, act=2, bm=b)
    z = _mm(z, Wr2, br2, act=2, bm=b)
    w3p = jnp.concatenate([Wr3, jnp.zeros((Wr3.shape[0], 127), jnp.float32)],
                          axis=1)
    b3p = jnp.concatenate([br3, jnp.zeros((127,), jnp.float32)])
    out = _mm(z, w3p, b3p, act=0, bm=b)
    return out[:, :1]


# speedup vs baseline: 5.7402x; 3.6845x over previous
import functools
import jax
import jax.numpy as jnp
from jax import lax
from jax.experimental import pallas as pl
from jax.experimental.pallas import tpu as pltpu
from jax.experimental.pallas import tpu_sc as plsc

_NC = 2
_NS = 16

B_K = 128
C1_K = 512


def _cdiv(a, b):
    return (a + b - 1) // b


def _mm_body(x_ref, w_ref, b_ref, o_ref, *, act):
    k = pl.program_id(1)

    @pl.when(k == 0)
    def _():
        o_ref[...] = jnp.zeros_like(o_ref)

    o_ref[...] += jnp.dot(x_ref[...], w_ref[...],
                          preferred_element_type=jnp.float32)

    @pl.when(k == pl.num_programs(1) - 1)
    def _():
        r = o_ref[...] + b_ref[...]
        if act == 1:
            r = jnp.maximum(r, 0.0)
        elif act == 2:
            r = jnp.where(r > 0, r, jnp.exp(jnp.minimum(r, 0.0)) - 1.0)
        o_ref[...] = r


def _mm(x, w, b, act=0, bm=None, bk=None):
    m, kk = x.shape
    _, n = w.shape
    if bm is None:
        bm = min(m, 1024)
    if bk is None:
        bk = kk if kk <= 2048 else 2048
    grid = (_cdiv(m, bm), _cdiv(kk, bk))
    return pl.pallas_call(
        functools.partial(_mm_body, act=act),
        grid=grid,
        in_specs=[
            pl.BlockSpec((bm, bk), lambda i, k: (i, k)),
            pl.BlockSpec((bk, n), lambda i, k: (k, 0)),
            pl.BlockSpec((1, n), lambda i, k: (0, 0)),
        ],
        out_specs=pl.BlockSpec((bm, n), lambda i, k: (i, 0)),
        out_shape=jax.ShapeDtypeStruct((m, n), jnp.float32),
        compiler_params=pltpu.CompilerParams(
            dimension_semantics=("parallel", "arbitrary")),
    )(x, w, b.reshape(1, -1))


def _bn_body(x_ref, a_ref, o_ref):
    xv = x_ref[...]
    a = a_ref[...]
    m = jnp.mean(xv, axis=0, keepdims=True)
    m2 = jnp.mean(xv * xv, axis=0, keepdims=True)
    mu = jnp.dot(m, a, preferred_element_type=jnp.float32)
    mu2 = jnp.dot(m2, a, preferred_element_type=jnp.float32)
    var = mu2 - mu * mu
    o_ref[...] = (xv - mu) * lax.rsqrt(var + 1e-5)


def _bn_pl(x):
    n, f = x.shape
    g = 128 // f
    xr = x.reshape(n // g, 128)
    lane = jnp.arange(128)
    a_mat = (lane[:, None] % f == lane[None, :] % f).astype(jnp.float32) / g
    out = pl.pallas_call(
        _bn_body,
        out_shape=jax.ShapeDtypeStruct(xr.shape, jnp.float32),
    )(xr, a_mat)
    return out.reshape(n, f)


def _sc_segsum(h, src, dst, k_chunk=80):
    n, d = h.shape
    e = src.shape[0]
    nw = _NC * _NS
    e_per = e // nw
    rowb = 80
    nb = _cdiv(n, rowb)
    assert e % nw == 0 and e_per % k_chunk == 0 and n % rowb == 0
    n_chunks = e_per // k_chunk
    mesh = plsc.VectorSubcoreMesh(core_axis_name="c", subcore_axis_name="s")

    @functools.partial(
        pl.kernel,
        out_type=jax.ShapeDtypeStruct((_NC, n, d), jnp.float32),
        mesh=mesh,
        scratch_types=[
            pltpu.VMEM_SHARED((n, d), jnp.float32),
            pltpu.VMEM((k_chunk,), jnp.int32),
            pltpu.VMEM((k_chunk,), jnp.int32),
            pltpu.VMEM((k_chunk, d), jnp.float32),
            pltpu.SemaphoreType.DMA,
        ],
    )
    def k(h_hbm, src_hbm, dst_hbm, out_hbm, acc, sidx, didx, rows, sem):
        cid = lax.axis_index("c")
        sid = lax.axis_index("s")
        for jj in range(_cdiv(nb, _NS)):
            j = jj * _NS + sid

            @pl.when(j < nb)
            def _():
                pltpu.sync_copy(h_hbm.at[pl.ds(j * rowb, rowb)],
                                acc.at[pl.ds(j * rowb, rowb)])
        plsc.subcore_barrier()
        wid = cid * _NS + sid

        def body(j, carry):
            base = wid * e_per + j * k_chunk
            pltpu.sync_copy(src_hbm.at[pl.ds(base, k_chunk)], sidx)
            pltpu.sync_copy(dst_hbm.at[pl.ds(base, k_chunk)], didx)
            pltpu.async_copy(h_hbm.at[sidx], rows, sem).wait()
            pltpu.sync_copy(rows, acc.at[didx], add=True)
            return carry

        lax.fori_loop(0, n_chunks, body, 0)
        plsc.subcore_barrier()
        for jj in range(_cdiv(nb, _NS)):
            j = jj * _NS + sid

            @pl.when(j < nb)
            def _():
                pltpu.sync_copy(acc.at[pl.ds(j * rowb, rowb)],
                                out_hbm.at[cid, pl.ds(j * rowb, rowb)])

    return k(h, src, dst)


def _mm_comb_body(p0_ref, p1_ref, h_ref, w_ref, b_ref, o_ref):
    x = p0_ref[0] + p1_ref[0] - h_ref[...]
    r = jnp.dot(x, w_ref[...], preferred_element_type=jnp.float32)
    o_ref[...] = jnp.maximum(r + b_ref[...], 0.0)


def _mm_comb(p, h, w, b, bm=1024):
    n, d = h.shape
    _, nn = w.shape
    grid = (_cdiv(n, bm),)
    return pl.pallas_call(
        _mm_comb_body,
        grid=grid,
        in_specs=[
            pl.BlockSpec((1, bm, d), lambda i: (0, i, 0)),
            pl.BlockSpec((1, bm, d), lambda i: (1, i, 0)),
            pl.BlockSpec((bm, d), lambda i: (i, 0)),
            pl.BlockSpec((d, nn), lambda i: (0, 0)),
            pl.BlockSpec((1, nn), lambda i: (0, 0)),
        ],
        out_specs=pl.BlockSpec((bm, nn), lambda i: (i, 0)),
        out_shape=jax.ShapeDtypeStruct((n, nn), jnp.float32),
        compiler_params=pltpu.CompilerParams(
            dimension_semantics=("parallel",)),
    )(p, p, h, w, b.reshape(1, -1))


def _sc_gat(tab, ed_col, src, dst, k_chunk=64):
    n = tab.shape[0]
    e = src.shape[0]
    nw = _NC * _NS
    e_per = e // nw
    assert e % nw == 0 and e_per % k_chunk == 0
    n_chunks = e_per // k_chunk
    rowb = 128
    nb = _cdiv(n, rowb)
    ngr = k_chunk // 16
    mesh = plsc.VectorSubcoreMesh(core_axis_name="c", subcore_axis_name="s")

    @functools.partial(
        pl.kernel,
        out_type=(jax.ShapeDtypeStruct((_NC, n, 16), jnp.float32),
                  jax.ShapeDtypeStruct((_NC, n), jnp.float32)),
        mesh=mesh,
        scratch_types=[
            pltpu.VMEM_SHARED((n, 16), jnp.float32),
            pltpu.VMEM_SHARED((n,), jnp.float32),
            pltpu.VMEM((k_chunk,), jnp.float32),
            pltpu.VMEM((k_chunk,), jnp.int32),
            pltpu.VMEM((k_chunk,), jnp.int32),
            pltpu.VMEM((k_chunk, 18), jnp.float32),
            pltpu.VMEM((k_chunk, 16), jnp.float32),
            pltpu.VMEM((k_chunk,), jnp.float32),
            pltpu.VMEM((rowb, 16), jnp.float32),
            pltpu.VMEM((rowb,), jnp.float32),
            pltpu.SemaphoreType.DMA,
        ],
        compiler_params=pltpu.CompilerParams(
            needs_layout_passes=False, use_tc_tiling_on_sc=False),
    )
    def k(tab_hbm, ed_hbm, src_hbm, dst_hbm,
          on_hbm, od_hbm, accn, accd, edb, sidx, didx, rows, pay, exb,
          zb, zdb, sem):
        cid = lax.axis_index("c")
        sid = lax.axis_index("s")
        wid = cid * _NS + sid
        z16 = jnp.zeros((16,), jnp.float32)
        for r in range(rowb):
            zb[r, :] = z16
        for r in range(rowb // 16):
            zdb[pl.ds(r * 16, 16)] = z16
        for jj in range(_cdiv(nb, _NS)):
            j = jj * _NS + sid

            @pl.when(j < nb)
            def _():
                pltpu.sync_copy(zb, accn.at[pl.ds(j * rowb, rowb)])
                pltpu.sync_copy(zdb, accd.at[pl.ds(j * rowb, rowb)])
        plsc.subcore_barrier()

        iota = lax.iota(jnp.int32, 16)
        col16 = jnp.full((16,), 16, jnp.int32)

        def body(j, carry):
            base = wid * e_per + j * k_chunk
            pltpu.sync_copy(src_hbm.at[pl.ds(base, k_chunk)], sidx)
            pltpu.sync_copy(dst_hbm.at[pl.ds(base, k_chunk)], didx)
            pltpu.async_copy(tab_hbm.at[sidx], rows, sem).wait()
            pltpu.async_copy(ed_hbm.at[didx], edb, sem).wait()
            for g in range(ngr):
                ed_vec = edb[pl.ds(g * 16, 16)]
                es_vec = plsc.load_gather(rows, [iota + g * 16, col16])
                ee = es_vec + ed_vec
                ee = jnp.where(ee >= 0, ee, 0.2 * ee)
                ex = jnp.exp(ee)
                exb[pl.ds(g * 16, 16)] = ex
                for lane in range(16):
                    j2 = g * 16 + lane
                    pay[j2, :] = rows[j2, pl.ds(0, 16)] * ex[lane]
            pltpu.sync_copy(pay, accn.at[didx], add=True)
            pltpu.sync_copy(exb, accd.at[didx], add=True)
            return carry

        lax.fori_loop(0, n_chunks, body, 0)
        plsc.subcore_barrier()
        for jj in range(_cdiv(nb, _NS)):
            j = jj * _NS + sid

            @pl.when(j < nb)
            def _():
                pltpu.sync_copy(accn.at[pl.ds(j * rowb, rowb)],
                                on_hbm.at[cid, pl.ds(j * rowb, rowb)])
                pltpu.sync_copy(accd.at[pl.ds(j * rowb, rowb)],
                                od_hbm.at[cid, pl.ds(j * rowb, rowb)])

    return k(tab, ed_col, src, dst)


def _gat_finish_body(h_ref, pn_ref, pd_ref, o_ref):
    hb = h_ref[...]
    h16 = hb[:, 0:16]
    ee = hb[:, 16:17] + hb[:, 17:18]
    ee = jnp.where(ee >= 0, ee, 0.2 * ee)
    w = jnp.exp(ee)
    num = pn_ref[0] + pn_ref[1] + w * h16
    den = pd_ref[0] + pd_ref[1] + w
    o_ref[...] = jnp.maximum(num / den, 0.0)


def _gat_finish(h_aug, pn, pd, bm=2048):
    n = h_aug.shape[0]
    pd3 = pd.reshape(_NC, n, 1)
    grid = (_cdiv(n, bm),)
    return pl.pallas_call(
        _gat_finish_body,
        grid=grid,
        in_specs=[
            pl.BlockSpec((bm, 18), lambda i: (i, 0)),
            pl.BlockSpec((_NC, bm, 16), lambda i: (0, i, 0)),
            pl.BlockSpec((_NC, bm, 1), lambda i: (0, i, 0)),
        ],
        out_specs=pl.BlockSpec((bm, 16), lambda i: (i, 0)),
        out_shape=jax.ShapeDtypeStruct((n, 16), jnp.float32),
        compiler_params=pltpu.CompilerParams(
            dimension_semantics=("parallel",)),
    )(h_aug, pn, pd3)


def _sc_pool(x, fidx, lidx, k_chunk=64):
    n_out = fidx.shape[0]
    nw = _NC * _NS
    per = n_out // nw
    assert n_out % nw == 0 and per % k_chunk == 0
    n_chunks = per // k_chunk
    mesh = plsc.VectorSubcoreMesh(core_axis_name="c", subcore_axis_name="s")

    @functools.partial(
        pl.kernel,
        out_type=jax.ShapeDtypeStruct((n_out, 16), jnp.float32),
        mesh=mesh,
        scratch_types=[
            pltpu.VMEM((k_chunk,), jnp.int32),
            pltpu.VMEM((k_chunk,), jnp.int32),
            pltpu.VMEM((k_chunk, 16), jnp.float32),
            pltpu.VMEM((k_chunk, 16), jnp.float32),
            pltpu.VMEM((k_chunk, 16), jnp.float32),
            pltpu.SemaphoreType.DMA,
        ],
        compiler_params=pltpu.CompilerParams(
            needs_layout_passes=False, use_tc_tiling_on_sc=False),
    )
    def k(x_hbm, f_hbm, l_hbm, out_hbm, fi, li, ra, rb, rc, sem):
        cid = lax.axis_index("c")
        sid = lax.axis_index("s")
        wid = cid * _NS + sid

        def body(j, carry):
            base = wid * per + j * k_chunk
            pltpu.sync_copy(f_hbm.at[pl.ds(base, k_chunk)], fi)
            pltpu.sync_copy(l_hbm.at[pl.ds(base, k_chunk)], li)
            pltpu.async_copy(x_hbm.at[fi], ra, sem).wait()
            pltpu.async_copy(x_hbm.at[li], rb, sem).wait()
            for j2 in range(k_chunk):
                rc[j2, :] = jnp.maximum(ra[j2, pl.ds(0, 16)],
                                        rb[j2, pl.ds(0, 16)])
            pltpu.sync_copy(rc, out_hbm.at[pl.ds(base, k_chunk)])
            return carry

        lax.fori_loop(0, n_chunks, body, 0)

    return k(x, fidx, lidx)


def _sc_segmax(x, starts, segs_per_tile, max_len):
    n, d = x.shape
    n_seg = starts.shape[0] - 1
    nw = _NC * _NS
    assert n_seg == nw * segs_per_tile
    nst = starts.shape[0]
    nst_pad = _cdiv(nst, 8) * 8
    starts_pad = jnp.concatenate(
        [starts, jnp.zeros((nst_pad - nst,), jnp.int32)])
    mesh = plsc.VectorSubcoreMesh(core_axis_name="c", subcore_axis_name="s")
    ml = _cdiv(max_len, 16) * 16

    @functools.partial(
        pl.kernel,
        out_type=jax.ShapeDtypeStruct((nw, segs_per_tile, d), jnp.float32),
        mesh=mesh,
        scratch_types=[
            pltpu.VMEM((nst_pad,), jnp.int32),
            pltpu.VMEM((ml,), jnp.int32),
            pltpu.VMEM((ml, d), jnp.float32),
            pltpu.VMEM((segs_per_tile, d), jnp.float32),
            pltpu.SemaphoreType.DMA,
        ],
        compiler_params=pltpu.CompilerParams(
            needs_layout_passes=False, use_tc_tiling_on_sc=False),
    )
    def k(x_hbm, st_hbm, out_hbm, stv, idxb, buf, ob, sem):
        cid = lax.axis_index("c")
        sid = lax.axis_index("s")
        wid = cid * _NS + sid
        pltpu.sync_copy(st_hbm, stv)
        iota = lax.iota(jnp.int32, 16)
        for t in range(segs_per_tile):
            seg = wid * segs_per_tile + t
            sv = plsc.load_gather(stv, [jnp.minimum(iota + seg, nst - 1)])
            s0 = sv[0]
            s1 = sv[1]
            for i in range(ml // 16):
                idxb[pl.ds(i * 16, 16)] = jnp.minimum(
                    iota + s0 + i * 16, n - 1)
            pltpu.async_copy(x_hbm.at[idxb], buf, sem).wait()
            for c in range(d // 16):
                acc = buf[0, pl.ds(c * 16, 16)]

                def red(r, a):
                    return jnp.maximum(a, buf[r, pl.ds(c * 16, 16)])

                acc = lax.fori_loop(1, s1 - s0, red, acc)
                ob[t, pl.ds(c * 16, 16)] = acc
        pltpu.sync_copy(ob, out_hbm.at[wid])

    return k(x, starts_pad).reshape(n_seg, d)


def kernel(drug_x, cell_x, W1, b1, W2, b2, W3, b3, Wd, bd, gW0, gas0, gad0,
           gW1, gas1, gad1, Wc1, bc1, Wc2, bc2, Wr1, br1, Wr2, br2, Wr3, br3,
           drug_edge_index, drug_batch, cell_edge_index, cluster0, cluster1):
    n_drug = drug_x.shape[0]
    n_cell = cell_x.shape[0]
    b = B_K
    genes = cluster0.shape[0]
    c0 = cluster1.shape[0]
    c1 = C1_K

    src, dst = drug_edge_index[0], drug_edge_index[1]

    h = drug_x
    src32 = src.astype(jnp.int32)
    dst32 = dst.astype(jnp.int32)
    reps = []
    for (W, bb) in ((W1, b1), (W2, b2), (W3, b3)):
        p = _sc_segsum(h, src32, dst32)
        h = _mm_comb(p, h, W, bb)
        reps.append(h)
    node_rep = jnp.concatenate(reps, axis=-1)
    starts = jnp.searchsorted(drug_batch.astype(jnp.int32),
                              jnp.arange(b + 1, dtype=jnp.int32)
                              ).astype(jnp.int32)
    x_drug = _sc_segmax(node_rep, starts, b // (_NC * _NS), 80)
    x_drug = _mm(x_drug, Wd, bd, act=1)

    csrc = cell_edge_index[0].astype(jnp.int32)
    cdst = cell_edge_index[1].astype(jnp.int32)
    va_s0 = gW0 @ gas0
    va_d0 = gW0 @ gad0
    W0_aug = jnp.concatenate([gW0, va_s0[:, None], va_d0[:, None]], axis=1)
    h_aug0 = _mm(cell_x, W0_aug, jnp.zeros((18,), jnp.float32))
    pn0, pd0 = _sc_gat(h_aug0, h_aug0[:, 17], csrc, cdst)
    h1 = _gat_finish(h_aug0, pn0, pd0)

    c0_t = cluster0.astype(jnp.int32)
    c1_t = cluster1.astype(jnp.int32)
    n1 = b * c0
    ar = jnp.arange(c0, dtype=jnp.int32)
    first0 = jnp.searchsorted(c0_t, ar).astype(jnp.int32)
    last0 = (jnp.searchsorted(c0_t, ar, side='right') - 1).astype(jnp.int32)
    boff = (jnp.arange(b, dtype=jnp.int32) * genes)[:, None]
    fidx0 = (boff + first0[None, :]).reshape(-1)
    lidx0 = (boff + last0[None, :]).reshape(-1)
    x = _sc_pool(h1, fidx0, lidx0)
    csrc2 = c0_t[csrc % genes] + (csrc // genes) * c0
    cdst2 = c0_t[cdst % genes] + (cdst // genes) * c0
    x = _bn_pl(x)

    va_s1 = gW1 @ gas1
    va_d1 = gW1 @ gad1
    W1_aug = jnp.concatenate([gW1, va_s1[:, None], va_d1[:, None]], axis=1)
    h_aug1 = _mm(x, W1_aug, jnp.zeros((18,), jnp.float32))
    pn1, pd1 = _sc_gat(h_aug1, h_aug1[:, 17], csrc2, cdst2)
    h2 = _gat_finish(h_aug1, pn1, pd1)

    n2 = b * c1
    ar1 = jnp.arange(c1, dtype=jnp.int32)
    first1 = jnp.searchsorted(c1_t, ar1).astype(jnp.int32)
    last1 = (jnp.searchsorted(c1_t, ar1, side='right') - 1).astype(jnp.int32)
    boff1 = (jnp.arange(b, dtype=jnp.int32) * c0)[:, None]
    fidx1 = (boff1 + first1[None, :]).reshape(-1)
    lidx1 = (boff1 + last1[None, :]).reshape(-1)
    x = _sc_pool(h2, fidx1, lidx1)
    x = _bn_pl(x)

    x_cell = x.reshape(b, c1 * 16)
    x_cell = _mm(x_cell, Wc1, bc1, act=1, bm=b)
    x_cell = _mm(x_cell, Wc2, bc2, act=1, bm=b)

    z = jnp.concatenate([x_drug, x_cell], axis=-1)
    z = _mm(z, Wr1, br1, act=2, bm=b)
    z = _mm(z, Wr2, br2, act=2, bm=b)
    w3p = jnp.concatenate([Wr3, jnp.zeros((Wr3.shape[0], 127), jnp.float32)],
                          axis=1)
    b3p = jnp.concatenate([br3, jnp.zeros((127,), jnp.float32)])
    out = _mm(z, w3p, b3p, act=0, bm=b)
    return out[:, :1]


# speedup vs baseline: 5.8952x; 1.0270x over previous
import functools
import jax
import jax.numpy as jnp
from jax import lax
from jax.experimental import pallas as pl
from jax.experimental.pallas import tpu as pltpu
from jax.experimental.pallas import tpu_sc as plsc

_NC = 2
_NS = 16

B_K = 128
C1_K = 512


def _cdiv(a, b):
    return (a + b - 1) // b


def _mm_body(x_ref, w_ref, b_ref, o_ref, *, act):
    k = pl.program_id(1)

    @pl.when(k == 0)
    def _():
        o_ref[...] = jnp.zeros_like(o_ref)

    o_ref[...] += jnp.dot(x_ref[...], w_ref[...],
                          preferred_element_type=jnp.float32)

    @pl.when(k == pl.num_programs(1) - 1)
    def _():
        r = o_ref[...] + b_ref[...]
        if act == 1:
            r = jnp.maximum(r, 0.0)
        elif act == 2:
            r = jnp.where(r > 0, r, jnp.exp(jnp.minimum(r, 0.0)) - 1.0)
        o_ref[...] = r


def _mm(x, w, b, act=0, bm=None, bk=None):
    m, kk = x.shape
    _, n = w.shape
    if bm is None:
        bm = min(m, 1024)
    if bk is None:
        bk = kk if kk <= 2048 else 2048
    grid = (_cdiv(m, bm), _cdiv(kk, bk))
    return pl.pallas_call(
        functools.partial(_mm_body, act=act),
        grid=grid,
        in_specs=[
            pl.BlockSpec((bm, bk), lambda i, k: (i, k)),
            pl.BlockSpec((bk, n), lambda i, k: (k, 0)),
            pl.BlockSpec((1, n), lambda i, k: (0, 0)),
        ],
        out_specs=pl.BlockSpec((bm, n), lambda i, k: (i, 0)),
        out_shape=jax.ShapeDtypeStruct((m, n), jnp.float32),
        compiler_params=pltpu.CompilerParams(
            dimension_semantics=("parallel", "arbitrary")),
    )(x, w, b.reshape(1, -1))


def _bn_body(x_ref, a_ref, o_ref):
    xv = x_ref[...]
    a = a_ref[...]
    m = jnp.mean(xv, axis=0, keepdims=True)
    m2 = jnp.mean(xv * xv, axis=0, keepdims=True)
    mu = jnp.dot(m, a, preferred_element_type=jnp.float32)
    mu2 = jnp.dot(m2, a, preferred_element_type=jnp.float32)
    var = mu2 - mu * mu
    o_ref[...] = (xv - mu) * lax.rsqrt(var + 1e-5)


def _bn_pl(x):
    n, f = x.shape
    g = 128 // f
    xr = x.reshape(n // g, 128)
    lane = jnp.arange(128)
    a_mat = (lane[:, None] % f == lane[None, :] % f).astype(jnp.float32) / g
    out = pl.pallas_call(
        _bn_body,
        out_shape=jax.ShapeDtypeStruct(xr.shape, jnp.float32),
    )(xr, a_mat)
    return out.reshape(n, f)


def _sc_segsum(h, src, dst, k_chunk=80):
    n, d = h.shape
    e = src.shape[0]
    nw = _NC * _NS
    e_per = e // nw
    rowb = 80
    nb = _cdiv(n, rowb)
    assert e % nw == 0 and e_per % k_chunk == 0 and n % rowb == 0
    n_chunks = e_per // k_chunk
    mesh = plsc.VectorSubcoreMesh(core_axis_name="c", subcore_axis_name="s")

    @functools.partial(
        pl.kernel,
        out_type=jax.ShapeDtypeStruct((_NC, n, d), jnp.float32),
        mesh=mesh,
        scratch_types=[
            pltpu.VMEM_SHARED((n, d), jnp.float32),
            pltpu.VMEM((k_chunk,), jnp.int32),
            pltpu.VMEM((k_chunk,), jnp.int32),
            pltpu.VMEM((k_chunk, d), jnp.float32),
            pltpu.SemaphoreType.DMA,
        ],
    )
    def k(h_hbm, src_hbm, dst_hbm, out_hbm, acc, sidx, didx, rows, sem):
        cid = lax.axis_index("c")
        sid = lax.axis_index("s")
        for jj in range(_cdiv(nb, _NS)):
            j = jj * _NS + sid

            @pl.when(j < nb)
            def _():
                pltpu.sync_copy(h_hbm.at[pl.ds(j * rowb, rowb)],
                                acc.at[pl.ds(j * rowb, rowb)])
        plsc.subcore_barrier()
        wid = cid * _NS + sid

        def body(j, carry):
            base = wid * e_per + j * k_chunk
            pltpu.sync_copy(src_hbm.at[pl.ds(base, k_chunk)], sidx)
            pltpu.sync_copy(dst_hbm.at[pl.ds(base, k_chunk)], didx)
            pltpu.async_copy(h_hbm.at[sidx], rows, sem).wait()
            pltpu.sync_copy(rows, acc.at[didx], add=True)
            return carry

        lax.fori_loop(0, n_chunks, body, 0)
        plsc.subcore_barrier()
        for jj in range(_cdiv(nb, _NS)):
            j = jj * _NS + sid

            @pl.when(j < nb)
            def _():
                pltpu.sync_copy(acc.at[pl.ds(j * rowb, rowb)],
                                out_hbm.at[cid, pl.ds(j * rowb, rowb)])

    return k(h, src, dst)


def _mm_comb_body(p0_ref, p1_ref, h_ref, w_ref, b_ref, o_ref):
    x = p0_ref[0] + p1_ref[0] - h_ref[...]
    r = jnp.dot(x, w_ref[...], preferred_element_type=jnp.float32)
    o_ref[...] = jnp.maximum(r + b_ref[...], 0.0)


def _mm_comb(p, h, w, b, bm=1024):
    n, d = h.shape
    _, nn = w.shape
    grid = (_cdiv(n, bm),)
    return pl.pallas_call(
        _mm_comb_body,
        grid=grid,
        in_specs=[
            pl.BlockSpec((1, bm, d), lambda i: (0, i, 0)),
            pl.BlockSpec((1, bm, d), lambda i: (1, i, 0)),
            pl.BlockSpec((bm, d), lambda i: (i, 0)),
            pl.BlockSpec((d, nn), lambda i: (0, 0)),
            pl.BlockSpec((1, nn), lambda i: (0, 0)),
        ],
        out_specs=pl.BlockSpec((bm, nn), lambda i: (i, 0)),
        out_shape=jax.ShapeDtypeStruct((n, nn), jnp.float32),
        compiler_params=pltpu.CompilerParams(
            dimension_semantics=("parallel",)),
    )(p, p, h, w, b.reshape(1, -1))


def _sc_gat(tab, ed_col, src, dst, k_chunk=64):
    n = tab.shape[0]
    e = src.shape[0]
    nw = _NC * _NS
    e_per = e // nw
    assert e % nw == 0 and e_per % k_chunk == 0
    n_chunks = e_per // k_chunk
    rowb = 128
    nb = _cdiv(n, rowb)
    ngr = k_chunk // 16
    mesh = plsc.VectorSubcoreMesh(core_axis_name="c", subcore_axis_name="s")

    @functools.partial(
        pl.kernel,
        out_type=(jax.ShapeDtypeStruct((_NC, n, 16), jnp.float32),
                  jax.ShapeDtypeStruct((_NC, n), jnp.float32)),
        mesh=mesh,
        scratch_types=[
            pltpu.VMEM_SHARED((n, 16), jnp.float32),
            pltpu.VMEM_SHARED((n,), jnp.float32),
            pltpu.VMEM((k_chunk,), jnp.float32),
            pltpu.VMEM((k_chunk,), jnp.int32),
            pltpu.VMEM((k_chunk,), jnp.int32),
            pltpu.VMEM((k_chunk, 18), jnp.float32),
            pltpu.VMEM((k_chunk, 16), jnp.float32),
            pltpu.VMEM((k_chunk,), jnp.float32),
            pltpu.VMEM((rowb, 16), jnp.float32),
            pltpu.VMEM((rowb,), jnp.float32),
            pltpu.SemaphoreType.DMA,
        ],
        compiler_params=pltpu.CompilerParams(
            needs_layout_passes=False, use_tc_tiling_on_sc=False),
    )
    def k(tab_hbm, ed_hbm, src_hbm, dst_hbm,
          on_hbm, od_hbm, accn, accd, edb, sidx, didx, rows, pay, exb,
          zb, zdb, sem):
        cid = lax.axis_index("c")
        sid = lax.axis_index("s")
        wid = cid * _NS + sid
        z16 = jnp.zeros((16,), jnp.float32)
        for r in range(rowb):
            zb[r, :] = z16
        for r in range(rowb // 16):
            zdb[pl.ds(r * 16, 16)] = z16
        for jj in range(_cdiv(nb, _NS)):
            j = jj * _NS + sid

            @pl.when(j < nb)
            def _():
                pltpu.sync_copy(zb, accn.at[pl.ds(j * rowb, rowb)])
                pltpu.sync_copy(zdb, accd.at[pl.ds(j * rowb, rowb)])
        plsc.subcore_barrier()

        iota = lax.iota(jnp.int32, 16)
        col16 = jnp.full((16,), 16, jnp.int32)

        def body(j, carry):
            base = wid * e_per + j * k_chunk
            pltpu.sync_copy(src_hbm.at[pl.ds(base, k_chunk)], sidx)
            pltpu.sync_copy(dst_hbm.at[pl.ds(base, k_chunk)], didx)
            cp1 = pltpu.async_copy(tab_hbm.at[sidx], rows, sem)
            cp2 = pltpu.async_copy(ed_hbm.at[didx], edb, sem)
            cp1.wait()
            cp2.wait()
            for g in range(ngr):
                ed_vec = edb[pl.ds(g * 16, 16)]
                es_vec = plsc.load_gather(rows, [iota + g * 16, col16])
                ee = es_vec + ed_vec
                ee = jnp.where(ee >= 0, ee, 0.2 * ee)
                ex = jnp.exp(ee)
                exb[pl.ds(g * 16, 16)] = ex
                for lane in range(16):
                    j2 = g * 16 + lane
                    pay[j2, :] = rows[j2, pl.ds(0, 16)] * ex[lane]
            pltpu.sync_copy(pay, accn.at[didx], add=True)
            pltpu.sync_copy(exb, accd.at[didx], add=True)
            return carry

        lax.fori_loop(0, n_chunks, body, 0)
        plsc.subcore_barrier()
        for jj in range(_cdiv(nb, _NS)):
            j = jj * _NS + sid

            @pl.when(j < nb)
            def _():
                pltpu.sync_copy(accn.at[pl.ds(j * rowb, rowb)],
                                on_hbm.at[cid, pl.ds(j * rowb, rowb)])
                pltpu.sync_copy(accd.at[pl.ds(j * rowb, rowb)],
                                od_hbm.at[cid, pl.ds(j * rowb, rowb)])

    return k(tab, ed_col, src, dst)


def _gat_finish_body(h_ref, pn_ref, pd_ref, o_ref):
    hb = h_ref[...]
    h16 = hb[:, 0:16]
    ee = hb[:, 16:17] + hb[:, 17:18]
    ee = jnp.where(ee >= 0, ee, 0.2 * ee)
    w = jnp.exp(ee)
    num = pn_ref[0] + pn_ref[1] + w * h16
    den = pd_ref[0] + pd_ref[1] + w
    o_ref[...] = jnp.maximum(num / den, 0.0)


def _gat_finish(h_aug, pn, pd, bm=2048):
    n = h_aug.shape[0]
    pd3 = pd.reshape(_NC, n, 1)
    grid = (_cdiv(n, bm),)
    return pl.pallas_call(
        _gat_finish_body,
        grid=grid,
        in_specs=[
            pl.BlockSpec((bm, 18), lambda i: (i, 0)),
            pl.BlockSpec((_NC, bm, 16), lambda i: (0, i, 0)),
            pl.BlockSpec((_NC, bm, 1), lambda i: (0, i, 0)),
        ],
        out_specs=pl.BlockSpec((bm, 16), lambda i: (i, 0)),
        out_shape=jax.ShapeDtypeStruct((n, 16), jnp.float32),
        compiler_params=pltpu.CompilerParams(
            dimension_semantics=("parallel",)),
    )(h_aug, pn, pd3)


def _sc_pool(x, fidx, lidx, k_chunk=64):
    n_out = fidx.shape[0]
    nw = _NC * _NS
    per = n_out // nw
    assert n_out % nw == 0 and per % k_chunk == 0
    n_chunks = per // k_chunk
    mesh = plsc.VectorSubcoreMesh(core_axis_name="c", subcore_axis_name="s")

    @functools.partial(
        pl.kernel,
        out_type=jax.ShapeDtypeStruct((n_out, 16), jnp.float32),
        mesh=mesh,
        scratch_types=[
            pltpu.VMEM((k_chunk,), jnp.int32),
            pltpu.VMEM((k_chunk,), jnp.int32),
            pltpu.VMEM((k_chunk, 16), jnp.float32),
            pltpu.VMEM((k_chunk, 16), jnp.float32),
            pltpu.VMEM((k_chunk, 16), jnp.float32),
            pltpu.SemaphoreType.DMA,
        ],
        compiler_params=pltpu.CompilerParams(
            needs_layout_passes=False, use_tc_tiling_on_sc=False),
    )
    def k(x_hbm, f_hbm, l_hbm, out_hbm, fi, li, ra, rb, rc, sem):
        cid = lax.axis_index("c")
        sid = lax.axis_index("s")
        wid = cid * _NS + sid

        def body(j, carry):
            base = wid * per + j * k_chunk
            pltpu.sync_copy(f_hbm.at[pl.ds(base, k_chunk)], fi)
            pltpu.sync_copy(l_hbm.at[pl.ds(base, k_chunk)], li)
            pltpu.async_copy(x_hbm.at[fi], ra, sem).wait()
            pltpu.async_copy(x_hbm.at[li], rb, sem).wait()
            for j2 in range(k_chunk):
                rc[j2, :] = jnp.maximum(ra[j2, pl.ds(0, 16)],
                                        rb[j2, pl.ds(0, 16)])
            pltpu.sync_copy(rc, out_hbm.at[pl.ds(base, k_chunk)])
            return carry

        lax.fori_loop(0, n_chunks, body, 0)

    return k(x, fidx, lidx)


def _sc_segmax(x, starts, segs_per_tile, max_len):
    n, d = x.shape
    n_seg = starts.shape[0] - 1
    nw = _NC * _NS
    assert n_seg == nw * segs_per_tile
    nst = starts.shape[0]
    nst_pad = _cdiv(nst, 8) * 8
    starts_pad = jnp.concatenate(
        [starts, jnp.zeros((nst_pad - nst,), jnp.int32)])
    mesh = plsc.VectorSubcoreMesh(core_axis_name="c", subcore_axis_name="s")
    ml = _cdiv(max_len, 16) * 16

    @functools.partial(
        pl.kernel,
        out_type=jax.ShapeDtypeStruct((nw, segs_per_tile, d), jnp.float32),
        mesh=mesh,
        scratch_types=[
            pltpu.VMEM((nst_pad,), jnp.int32),
            pltpu.VMEM((ml,), jnp.int32),
            pltpu.VMEM((ml, d), jnp.float32),
            pltpu.VMEM((segs_per_tile, d), jnp.float32),
            pltpu.SemaphoreType.DMA,
        ],
        compiler_params=pltpu.CompilerParams(
            needs_layout_passes=False, use_tc_tiling_on_sc=False),
    )
    def k(x_hbm, st_hbm, out_hbm, stv, idxb, buf, ob, sem):
        cid = lax.axis_index("c")
        sid = lax.axis_index("s")
        wid = cid * _NS + sid
        pltpu.sync_copy(st_hbm, stv)
        iota = lax.iota(jnp.int32, 16)
        for t in range(segs_per_tile):
            seg = wid * segs_per_tile + t
            sv = plsc.load_gather(stv, [jnp.minimum(iota + seg, nst - 1)])
            s0 = sv[0]
            s1 = sv[1]
            for i in range(ml // 16):
                idxb[pl.ds(i * 16, 16)] = jnp.minimum(
                    iota + s0 + i * 16, n - 1)
            pltpu.async_copy(x_hbm.at[idxb], buf, sem).wait()
            for c in range(d // 16):
                acc = buf[0, pl.ds(c * 16, 16)]

                def red(r, a):
                    return jnp.maximum(a, buf[r, pl.ds(c * 16, 16)])

                acc = lax.fori_loop(1, s1 - s0, red, acc)
                ob[t, pl.ds(c * 16, 16)] = acc
        pltpu.sync_copy(ob, out_hbm.at[wid])

    return k(x, starts_pad).reshape(n_seg, d)


def kernel(drug_x, cell_x, W1, b1, W2, b2, W3, b3, Wd, bd, gW0, gas0, gad0,
           gW1, gas1, gad1, Wc1, bc1, Wc2, bc2, Wr1, br1, Wr2, br2, Wr3, br3,
           drug_edge_index, drug_batch, cell_edge_index, cluster0, cluster1):
    n_drug = drug_x.shape[0]
    n_cell = cell_x.shape[0]
    b = B_K
    genes = cluster0.shape[0]
    c0 = cluster1.shape[0]
    c1 = C1_K

    src, dst = drug_edge_index[0], drug_edge_index[1]

    h = drug_x
    src32 = src.astype(jnp.int32)
    dst32 = dst.astype(jnp.int32)
    reps = []
    for (W, bb) in ((W1, b1), (W2, b2), (W3, b3)):
        p = _sc_segsum(h, src32, dst32)
        h = _mm_comb(p, h, W, bb)
        reps.append(h)
    node_rep = jnp.concatenate(reps, axis=-1)
    starts = jnp.searchsorted(drug_batch.astype(jnp.int32),
                              jnp.arange(b + 1, dtype=jnp.int32)
                              ).astype(jnp.int32)
    x_drug = _sc_segmax(node_rep, starts, b // (_NC * _NS), 80)
    x_drug = _mm(x_drug, Wd, bd, act=1)

    csrc = cell_edge_index[0].astype(jnp.int32)
    cdst = cell_edge_index[1].astype(jnp.int32)
    va_s0 = gW0 @ gas0
    va_d0 = gW0 @ gad0
    W0_aug = jnp.concatenate([gW0, va_s0[:, None], va_d0[:, None]], axis=1)
    h_aug0 = _mm(cell_x, W0_aug, jnp.zeros((18,), jnp.float32))
    pn0, pd0 = _sc_gat(h_aug0, h_aug0[:, 17], csrc, cdst)
    h1 = _gat_finish(h_aug0, pn0, pd0)

    c0_t = cluster0.astype(jnp.int32)
    c1_t = cluster1.astype(jnp.int32)
    n1 = b * c0
    ar = jnp.arange(c0, dtype=jnp.int32)
    first0 = jnp.searchsorted(c0_t, ar).astype(jnp.int32)
    last0 = (jnp.searchsorted(c0_t, ar, side='right') - 1).astype(jnp.int32)
    boff = (jnp.arange(b, dtype=jnp.int32) * genes)[:, None]
    fidx0 = (boff + first0[None, :]).reshape(-1)
    lidx0 = (boff + last0[None, :]).reshape(-1)
    x = _sc_pool(h1, fidx0, lidx0)
    csrc2 = c0_t[csrc % genes] + (csrc // genes) * c0
    cdst2 = c0_t[cdst % genes] + (cdst // genes) * c0
    x = _bn_pl(x)

    va_s1 = gW1 @ gas1
    va_d1 = gW1 @ gad1
    W1_aug = jnp.concatenate([gW1, va_s1[:, None], va_d1[:, None]], axis=1)
    h_aug1 = _mm(x, W1_aug, jnp.zeros((18,), jnp.float32))
    pn1, pd1 = _sc_gat(h_aug1, h_aug1[:, 17], csrc2, cdst2)
    h2 = _gat_finish(h_aug1, pn1, pd1)

    n2 = b * c1
    ar1 = jnp.arange(c1, dtype=jnp.int32)
    first1 = jnp.searchsorted(c1_t, ar1).astype(jnp.int32)
    last1 = (jnp.searchsorted(c1_t, ar1, side='right') - 1).astype(jnp.int32)
    boff1 = (jnp.arange(b, dtype=jnp.int32) * c0)[:, None]
    fidx1 = (boff1 + first1[None, :]).reshape(-1)
    lidx1 = (boff1 + last1[None, :]).reshape(-1)
    x = _sc_pool(h2, fidx1, lidx1)
    x = _bn_pl(x)

    x_cell = x.reshape(b, c1 * 16)
    x_cell = _mm(x_cell, Wc1, bc1, act=1, bm=b)
    x_cell = _mm(x_cell, Wc2, bc2, act=1, bm=b)

    z = jnp.concatenate([x_drug, x_cell], axis=-1)
    z = _mm(z, Wr1, br1, act=2, bm=b)
    z = _mm(z, Wr2, br2, act=2, bm=b)
    w3p = jnp.concatenate([Wr3, jnp.zeros((Wr3.shape[0], 127), jnp.float32)],
                          axis=1)
    b3p = jnp.concatenate([br3, jnp.zeros((127,), jnp.float32)])
    out = _mm(z, w3p, b3p, act=0, bm=b)
    return out[:, :1]


# speedup vs baseline: 6.0405x; 1.0246x over previous
import functools
import jax
import jax.numpy as jnp
from jax import lax
from jax.experimental import pallas as pl
from jax.experimental.pallas import tpu as pltpu
from jax.experimental.pallas import tpu_sc as plsc

_NC = 2
_NS = 16

B_K = 128
C1_K = 512


def _cdiv(a, b):
    return (a + b - 1) // b


def _mm_body(x_ref, w_ref, b_ref, o_ref, *, act):
    k = pl.program_id(1)

    @pl.when(k == 0)
    def _():
        o_ref[...] = jnp.zeros_like(o_ref)

    o_ref[...] += jnp.dot(x_ref[...], w_ref[...],
                          preferred_element_type=jnp.float32)

    @pl.when(k == pl.num_programs(1) - 1)
    def _():
        r = o_ref[...] + b_ref[...]
        if act == 1:
            r = jnp.maximum(r, 0.0)
        elif act == 2:
            r = jnp.where(r > 0, r, jnp.exp(jnp.minimum(r, 0.0)) - 1.0)
        o_ref[...] = r


def _mm(x, w, b, act=0, bm=None, bk=None):
    m, kk = x.shape
    _, n = w.shape
    if bm is None:
        bm = min(m, 1024)
    if bk is None:
        bk = kk if kk <= 2048 else 2048
    grid = (_cdiv(m, bm), _cdiv(kk, bk))
    return pl.pallas_call(
        functools.partial(_mm_body, act=act),
        grid=grid,
        in_specs=[
            pl.BlockSpec((bm, bk), lambda i, k: (i, k)),
            pl.BlockSpec((bk, n), lambda i, k: (k, 0)),
            pl.BlockSpec((1, n), lambda i, k: (0, 0)),
        ],
        out_specs=pl.BlockSpec((bm, n), lambda i, k: (i, 0)),
        out_shape=jax.ShapeDtypeStruct((m, n), jnp.float32),
        compiler_params=pltpu.CompilerParams(
            dimension_semantics=("parallel", "arbitrary")),
    )(x, w, b.reshape(1, -1))


def _bn_body(x_ref, a_ref, o_ref):
    xv = x_ref[...]
    a = a_ref[...]
    m = jnp.mean(xv, axis=0, keepdims=True)
    m2 = jnp.mean(xv * xv, axis=0, keepdims=True)
    mu = jnp.dot(m, a, preferred_element_type=jnp.float32)
    mu2 = jnp.dot(m2, a, preferred_element_type=jnp.float32)
    var = mu2 - mu * mu
    o_ref[...] = (xv - mu) * lax.rsqrt(var + 1e-5)


def _bn_pl(x):
    n, f = x.shape
    g = 128 // f
    xr = x.reshape(n // g, 128)
    lane = jnp.arange(128)
    a_mat = (lane[:, None] % f == lane[None, :] % f).astype(jnp.float32) / g
    out = pl.pallas_call(
        _bn_body,
        out_shape=jax.ShapeDtypeStruct(xr.shape, jnp.float32),
    )(xr, a_mat)
    return out.reshape(n, f)


def _sc_segsum(h, src, dst, k_chunk=80):
    n, d = h.shape
    e = src.shape[0]
    nw = _NC * _NS
    e_per = e // nw
    rowb = 80
    nb = _cdiv(n, rowb)
    assert e % nw == 0 and e_per % k_chunk == 0 and n % rowb == 0
    n_chunks = e_per // k_chunk
    mesh = plsc.VectorSubcoreMesh(core_axis_name="c", subcore_axis_name="s")

    @functools.partial(
        pl.kernel,
        out_type=jax.ShapeDtypeStruct((_NC, n, d), jnp.float32),
        mesh=mesh,
        scratch_types=[
            pltpu.VMEM_SHARED((n, d), jnp.float32),
            pltpu.VMEM((k_chunk,), jnp.int32),
            pltpu.VMEM((k_chunk,), jnp.int32),
            pltpu.VMEM((k_chunk, d), jnp.float32),
            pltpu.SemaphoreType.DMA,
        ],
    )
    def k(h_hbm, src_hbm, dst_hbm, out_hbm, acc, sidx, didx, rows, sem):
        cid = lax.axis_index("c")
        sid = lax.axis_index("s")
        for jj in range(_cdiv(nb, _NS)):
            j = jj * _NS + sid

            @pl.when(j < nb)
            def _():
                pltpu.sync_copy(h_hbm.at[pl.ds(j * rowb, rowb)],
                                acc.at[pl.ds(j * rowb, rowb)])
        plsc.subcore_barrier()
        wid = cid * _NS + sid

        def body(j, carry):
            base = wid * e_per + j * k_chunk
            ci = pltpu.async_copy(src_hbm.at[pl.ds(base, k_chunk)], sidx, sem)
            cd = pltpu.async_copy(dst_hbm.at[pl.ds(base, k_chunk)], didx, sem)
            ci.wait()
            cr = pltpu.async_copy(h_hbm.at[sidx], rows, sem)
            cd.wait()
            cr.wait()
            pltpu.sync_copy(rows, acc.at[didx], add=True)
            return carry

        lax.fori_loop(0, n_chunks, body, 0)
        plsc.subcore_barrier()
        for jj in range(_cdiv(nb, _NS)):
            j = jj * _NS + sid

            @pl.when(j < nb)
            def _():
                pltpu.sync_copy(acc.at[pl.ds(j * rowb, rowb)],
                                out_hbm.at[cid, pl.ds(j * rowb, rowb)])

    return k(h, src, dst)


def _mm_comb_body(p0_ref, p1_ref, h_ref, w_ref, b_ref, o_ref):
    x = p0_ref[0] + p1_ref[0] - h_ref[...]
    r = jnp.dot(x, w_ref[...], preferred_element_type=jnp.float32)
    o_ref[...] = jnp.maximum(r + b_ref[...], 0.0)


def _mm_comb(p, h, w, b, bm=1024):
    n, d = h.shape
    _, nn = w.shape
    grid = (_cdiv(n, bm),)
    return pl.pallas_call(
        _mm_comb_body,
        grid=grid,
        in_specs=[
            pl.BlockSpec((1, bm, d), lambda i: (0, i, 0)),
            pl.BlockSpec((1, bm, d), lambda i: (1, i, 0)),
            pl.BlockSpec((bm, d), lambda i: (i, 0)),
            pl.BlockSpec((d, nn), lambda i: (0, 0)),
            pl.BlockSpec((1, nn), lambda i: (0, 0)),
        ],
        out_specs=pl.BlockSpec((bm, nn), lambda i: (i, 0)),
        out_shape=jax.ShapeDtypeStruct((n, nn), jnp.float32),
        compiler_params=pltpu.CompilerParams(
            dimension_semantics=("parallel",)),
    )(p, p, h, w, b.reshape(1, -1))


def _sc_gat(tab, ed_col, src, dst, k_chunk=64):
    n = tab.shape[0]
    e = src.shape[0]
    nw = _NC * _NS
    e_per = e // nw
    assert e % nw == 0 and e_per % k_chunk == 0
    n_chunks = e_per // k_chunk
    rowb = 128
    nb = _cdiv(n, rowb)
    ngr = k_chunk // 16
    mesh = plsc.VectorSubcoreMesh(core_axis_name="c", subcore_axis_name="s")

    @functools.partial(
        pl.kernel,
        out_type=(jax.ShapeDtypeStruct((_NC, n, 16), jnp.float32),
                  jax.ShapeDtypeStruct((_NC, n), jnp.float32)),
        mesh=mesh,
        scratch_types=[
            pltpu.VMEM_SHARED((n, 16), jnp.float32),
            pltpu.VMEM_SHARED((n,), jnp.float32),
            pltpu.VMEM((k_chunk,), jnp.float32),
            pltpu.VMEM((k_chunk,), jnp.int32),
            pltpu.VMEM((k_chunk,), jnp.int32),
            pltpu.VMEM((k_chunk, 18), jnp.float32),
            pltpu.VMEM((k_chunk, 16), jnp.float32),
            pltpu.VMEM((k_chunk,), jnp.float32),
            pltpu.VMEM((rowb, 16), jnp.float32),
            pltpu.VMEM((rowb,), jnp.float32),
            pltpu.SemaphoreType.DMA,
        ],
        compiler_params=pltpu.CompilerParams(
            needs_layout_passes=False, use_tc_tiling_on_sc=False),
    )
    def k(tab_hbm, ed_hbm, src_hbm, dst_hbm,
          on_hbm, od_hbm, accn, accd, edb, sidx, didx, rows, pay, exb,
          zb, zdb, sem):
        cid = lax.axis_index("c")
        sid = lax.axis_index("s")
        wid = cid * _NS + sid
        z16 = jnp.zeros((16,), jnp.float32)
        for r in range(rowb):
            zb[r, :] = z16
        for r in range(rowb // 16):
            zdb[pl.ds(r * 16, 16)] = z16
        for jj in range(_cdiv(nb, _NS)):
            j = jj * _NS + sid

            @pl.when(j < nb)
            def _():
                pltpu.sync_copy(zb, accn.at[pl.ds(j * rowb, rowb)])
                pltpu.sync_copy(zdb, accd.at[pl.ds(j * rowb, rowb)])
        plsc.subcore_barrier()

        iota = lax.iota(jnp.int32, 16)
        col16 = jnp.full((16,), 16, jnp.int32)

        def body(j, carry):
            base = wid * e_per + j * k_chunk
            ci = pltpu.async_copy(src_hbm.at[pl.ds(base, k_chunk)], sidx, sem)
            cd = pltpu.async_copy(dst_hbm.at[pl.ds(base, k_chunk)], didx, sem)
            ci.wait()
            cp1 = pltpu.async_copy(tab_hbm.at[sidx], rows, sem)
            cd.wait()
            cp2 = pltpu.async_copy(ed_hbm.at[didx], edb, sem)
            cp1.wait()
            cp2.wait()
            for g in range(ngr):
                ed_vec = edb[pl.ds(g * 16, 16)]
                es_vec = plsc.load_gather(rows, [iota + g * 16, col16])
                ee = es_vec + ed_vec
                ee = jnp.where(ee >= 0, ee, 0.2 * ee)
                ex = jnp.exp(ee)
                exb[pl.ds(g * 16, 16)] = ex
                for lane in range(16):
                    j2 = g * 16 + lane
                    pay[j2, :] = rows[j2, pl.ds(0, 16)] * ex[lane]
            pltpu.sync_copy(pay, accn.at[didx], add=True)
            pltpu.sync_copy(exb, accd.at[didx], add=True)
            return carry

        lax.fori_loop(0, n_chunks, body, 0)
        plsc.subcore_barrier()
        for jj in range(_cdiv(nb, _NS)):
            j = jj * _NS + sid

            @pl.when(j < nb)
            def _():
                pltpu.sync_copy(accn.at[pl.ds(j * rowb, rowb)],
                                on_hbm.at[cid, pl.ds(j * rowb, rowb)])
                pltpu.sync_copy(accd.at[pl.ds(j * rowb, rowb)],
                                od_hbm.at[cid, pl.ds(j * rowb, rowb)])

    return k(tab, ed_col, src, dst)


def _gat_finish_body(h_ref, pn_ref, pd_ref, o_ref):
    hb = h_ref[...]
    h16 = hb[:, 0:16]
    ee = hb[:, 16:17] + hb[:, 17:18]
    ee = jnp.where(ee >= 0, ee, 0.2 * ee)
    w = jnp.exp(ee)
    num = pn_ref[0] + pn_ref[1] + w * h16
    den = pd_ref[0] + pd_ref[1] + w
    o_ref[...] = jnp.maximum(num / den, 0.0)


def _gat_finish(h_aug, pn, pd, bm=2048):
    n = h_aug.shape[0]
    pd3 = pd.reshape(_NC, n, 1)
    grid = (_cdiv(n, bm),)
    return pl.pallas_call(
        _gat_finish_body,
        grid=grid,
        in_specs=[
            pl.BlockSpec((bm, 18), lambda i: (i, 0)),
            pl.BlockSpec((_NC, bm, 16), lambda i: (0, i, 0)),
            pl.BlockSpec((_NC, bm, 1), lambda i: (0, i, 0)),
        ],
        out_specs=pl.BlockSpec((bm, 16), lambda i: (i, 0)),
        out_shape=jax.ShapeDtypeStruct((n, 16), jnp.float32),
        compiler_params=pltpu.CompilerParams(
            dimension_semantics=("parallel",)),
    )(h_aug, pn, pd3)


def _sc_pool(x, fidx, lidx, k_chunk=64):
    n_out = fidx.shape[0]
    nw = _NC * _NS
    per = n_out // nw
    assert n_out % nw == 0 and per % k_chunk == 0
    n_chunks = per // k_chunk
    mesh = plsc.VectorSubcoreMesh(core_axis_name="c", subcore_axis_name="s")

    @functools.partial(
        pl.kernel,
        out_type=jax.ShapeDtypeStruct((n_out, 16), jnp.float32),
        mesh=mesh,
        scratch_types=[
            pltpu.VMEM((k_chunk,), jnp.int32),
            pltpu.VMEM((k_chunk,), jnp.int32),
            pltpu.VMEM((k_chunk, 16), jnp.float32),
            pltpu.VMEM((k_chunk, 16), jnp.float32),
            pltpu.VMEM((k_chunk, 16), jnp.float32),
            pltpu.SemaphoreType.DMA,
        ],
        compiler_params=pltpu.CompilerParams(
            needs_layout_passes=False, use_tc_tiling_on_sc=False),
    )
    def k(x_hbm, f_hbm, l_hbm, out_hbm, fi, li, ra, rb, rc, sem):
        cid = lax.axis_index("c")
        sid = lax.axis_index("s")
        wid = cid * _NS + sid

        def body(j, carry):
            base = wid * per + j * k_chunk
            pltpu.sync_copy(f_hbm.at[pl.ds(base, k_chunk)], fi)
            pltpu.sync_copy(l_hbm.at[pl.ds(base, k_chunk)], li)
            pltpu.async_copy(x_hbm.at[fi], ra, sem).wait()
            pltpu.async_copy(x_hbm.at[li], rb, sem).wait()
            for j2 in range(k_chunk):
                rc[j2, :] = jnp.maximum(ra[j2, pl.ds(0, 16)],
                                        rb[j2, pl.ds(0, 16)])
            pltpu.sync_copy(rc, out_hbm.at[pl.ds(base, k_chunk)])
            return carry

        lax.fori_loop(0, n_chunks, body, 0)

    return k(x, fidx, lidx)


def _sc_segmax(x, starts, segs_per_tile, max_len):
    n, d = x.shape
    n_seg = starts.shape[0] - 1
    nw = _NC * _NS
    assert n_seg == nw * segs_per_tile
    nst = starts.shape[0]
    nst_pad = _cdiv(nst, 8) * 8
    starts_pad = jnp.concatenate(
        [starts, jnp.zeros((nst_pad - nst,), jnp.int32)])
    mesh = plsc.VectorSubcoreMesh(core_axis_name="c", subcore_axis_name="s")
    ml = _cdiv(max_len, 16) * 16

    @functools.partial(
        pl.kernel,
        out_type=jax.ShapeDtypeStruct((nw, segs_per_tile, d), jnp.float32),
        mesh=mesh,
        scratch_types=[
            pltpu.VMEM((nst_pad,), jnp.int32),
            pltpu.VMEM((ml,), jnp.int32),
            pltpu.VMEM((ml, d), jnp.float32),
            pltpu.VMEM((segs_per_tile, d), jnp.float32),
            pltpu.SemaphoreType.DMA,
        ],
        compiler_params=pltpu.CompilerParams(
            needs_layout_passes=False, use_tc_tiling_on_sc=False),
    )
    def k(x_hbm, st_hbm, out_hbm, stv, idxb, buf, ob, sem):
        cid = lax.axis_index("c")
        sid = lax.axis_index("s")
        wid = cid * _NS + sid
        pltpu.sync_copy(st_hbm, stv)
        iota = lax.iota(jnp.int32, 16)
        for t in range(segs_per_tile):
            seg = wid * segs_per_tile + t
            sv = plsc.load_gather(stv, [jnp.minimum(iota + seg, nst - 1)])
            s0 = sv[0]
            s1 = sv[1]
            for i in range(ml // 16):
                idxb[pl.ds(i * 16, 16)] = jnp.minimum(
                    iota + s0 + i * 16, n - 1)
            pltpu.async_copy(x_hbm.at[idxb], buf, sem).wait()
            for c in range(d // 16):
                acc = buf[0, pl.ds(c * 16, 16)]

                def red(r, a):
                    return jnp.maximum(a, buf[r, pl.ds(c * 16, 16)])

                acc = lax.fori_loop(1, s1 - s0, red, acc)
                ob[t, pl.ds(c * 16, 16)] = acc
        pltpu.sync_copy(ob, out_hbm.at[wid])

    return k(x, starts_pad).reshape(n_seg, d)


def kernel(drug_x, cell_x, W1, b1, W2, b2, W3, b3, Wd, bd, gW0, gas0, gad0,
           gW1, gas1, gad1, Wc1, bc1, Wc2, bc2, Wr1, br1, Wr2, br2, Wr3, br3,
           drug_edge_index, drug_batch, cell_edge_index, cluster0, cluster1):
    n_drug = drug_x.shape[0]
    n_cell = cell_x.shape[0]
    b = B_K
    genes = cluster0.shape[0]
    c0 = cluster1.shape[0]
    c1 = C1_K

    src, dst = drug_edge_index[0], drug_edge_index[1]

    h = drug_x
    src32 = src.astype(jnp.int32)
    dst32 = dst.astype(jnp.int32)
    reps = []
    for (W, bb) in ((W1, b1), (W2, b2), (W3, b3)):
        p = _sc_segsum(h, src32, dst32)
        h = _mm_comb(p, h, W, bb)
        reps.append(h)
    node_rep = jnp.concatenate(reps, axis=-1)
    starts = jnp.searchsorted(drug_batch.astype(jnp.int32),
                              jnp.arange(b + 1, dtype=jnp.int32)
                              ).astype(jnp.int32)
    x_drug = _sc_segmax(node_rep, starts, b // (_NC * _NS), 80)
    x_drug = _mm(x_drug, Wd, bd, act=1)

    csrc = cell_edge_index[0].astype(jnp.int32)
    cdst = cell_edge_index[1].astype(jnp.int32)
    va_s0 = gW0 @ gas0
    va_d0 = gW0 @ gad0
    W0_aug = jnp.concatenate([gW0, va_s0[:, None], va_d0[:, None]], axis=1)
    h_aug0 = _mm(cell_x, W0_aug, jnp.zeros((18,), jnp.float32))
    pn0, pd0 = _sc_gat(h_aug0, h_aug0[:, 17], csrc, cdst)
    h1 = _gat_finish(h_aug0, pn0, pd0)

    c0_t = cluster0.astype(jnp.int32)
    c1_t = cluster1.astype(jnp.int32)
    n1 = b * c0
    ar = jnp.arange(c0, dtype=jnp.int32)
    first0 = jnp.searchsorted(c0_t, ar).astype(jnp.int32)
    last0 = (jnp.searchsorted(c0_t, ar, side='right') - 1).astype(jnp.int32)
    boff = (jnp.arange(b, dtype=jnp.int32) * genes)[:, None]
    fidx0 = (boff + first0[None, :]).reshape(-1)
    lidx0 = (boff + last0[None, :]).reshape(-1)
    x = _sc_pool(h1, fidx0, lidx0)
    csrc2 = c0_t[csrc % genes] + (csrc // genes) * c0
    cdst2 = c0_t[cdst % genes] + (cdst // genes) * c0
    x = _bn_pl(x)

    va_s1 = gW1 @ gas1
    va_d1 = gW1 @ gad1
    W1_aug = jnp.concatenate([gW1, va_s1[:, None], va_d1[:, None]], axis=1)
    h_aug1 = _mm(x, W1_aug, jnp.zeros((18,), jnp.float32))
    pn1, pd1 = _sc_gat(h_aug1, h_aug1[:, 17], csrc2, cdst2)
    h2 = _gat_finish(h_aug1, pn1, pd1)

    n2 = b * c1
    ar1 = jnp.arange(c1, dtype=jnp.int32)
    first1 = jnp.searchsorted(c1_t, ar1).astype(jnp.int32)
    last1 = (jnp.searchsorted(c1_t, ar1, side='right') - 1).astype(jnp.int32)
    boff1 = (jnp.arange(b, dtype=jnp.int32) * c0)[:, None]
    fidx1 = (boff1 + first1[None, :]).reshape(-1)
    lidx1 = (boff1 + last1[None, :]).reshape(-1)
    x = _sc_pool(h2, fidx1, lidx1)
    x = _bn_pl(x)

    x_cell = x.reshape(b, c1 * 16)
    x_cell = _mm(x_cell, Wc1, bc1, act=1, bm=b)
    x_cell = _mm(x_cell, Wc2, bc2, act=1, bm=b)

    z = jnp.concatenate([x_drug, x_cell], axis=-1)
    z = _mm(z, Wr1, br1, act=2, bm=b)
    z = _mm(z, Wr2, br2, act=2, bm=b)
    w3p = jnp.concatenate([Wr3, jnp.zeros((Wr3.shape[0], 127), jnp.float32)],
                          axis=1)
    b3p = jnp.concatenate([br3, jnp.zeros((127,), jnp.float32)])
    out = _mm(z, w3p, b3p, act=0, bm=b)
    return out[:, :1]


# speedup vs baseline: 6.0532x; 1.0021x over previous
import functools
import jax
import jax.numpy as jnp
from jax import lax
from jax.experimental import pallas as pl
from jax.experimental.pallas import tpu as pltpu
from jax.experimental.pallas import tpu_sc as plsc

_NC = 2
_NS = 16

B_K = 128
C1_K = 512


def _cdiv(a, b):
    return (a + b - 1) // b


def _mm_body(x_ref, w_ref, b_ref, o_ref, *, act):
    k = pl.program_id(1)

    @pl.when(k == 0)
    def _():
        o_ref[...] = jnp.zeros_like(o_ref)

    o_ref[...] += jnp.dot(x_ref[...], w_ref[...],
                          preferred_element_type=jnp.float32)

    @pl.when(k == pl.num_programs(1) - 1)
    def _():
        r = o_ref[...] + b_ref[...]
        if act == 1:
            r = jnp.maximum(r, 0.0)
        elif act == 2:
            r = jnp.where(r > 0, r, jnp.exp(jnp.minimum(r, 0.0)) - 1.0)
        o_ref[...] = r


def _mm(x, w, b, act=0, bm=None, bk=None):
    m, kk = x.shape
    _, n = w.shape
    if bm is None:
        bm = min(m, 1024)
    if bk is None:
        bk = kk if kk <= 2048 else 2048
    grid = (_cdiv(m, bm), _cdiv(kk, bk))
    return pl.pallas_call(
        functools.partial(_mm_body, act=act),
        grid=grid,
        in_specs=[
            pl.BlockSpec((bm, bk), lambda i, k: (i, k)),
            pl.BlockSpec((bk, n), lambda i, k: (k, 0)),
            pl.BlockSpec((1, n), lambda i, k: (0, 0)),
        ],
        out_specs=pl.BlockSpec((bm, n), lambda i, k: (i, 0)),
        out_shape=jax.ShapeDtypeStruct((m, n), jnp.float32),
        compiler_params=pltpu.CompilerParams(
            dimension_semantics=("parallel", "arbitrary")),
    )(x, w, b.reshape(1, -1))


def _bn_body(x_ref, a_ref, o_ref):
    xv = x_ref[...]
    a = a_ref[...]
    m = jnp.mean(xv, axis=0, keepdims=True)
    mu = jnp.dot(m, a, preferred_element_type=jnp.float32)
    d = xv - mu
    v = jnp.mean(d * d, axis=0, keepdims=True)
    var = jnp.dot(v, a, preferred_element_type=jnp.float32)
    o_ref[...] = d * lax.rsqrt(var + 1e-5)


def _bn_pl(x):
    n, f = x.shape
    g = 128 // f
    xr = x.reshape(n // g, 128)
    lane = jnp.arange(128)
    a_mat = (lane[:, None] % f == lane[None, :] % f).astype(jnp.float32) / g
    out = pl.pallas_call(
        _bn_body,
        out_shape=jax.ShapeDtypeStruct(xr.shape, jnp.float32),
    )(xr, a_mat)
    return out.reshape(n, f)


def _sc_segsum(h, src, dst, k_chunk=80):
    n, d = h.shape
    e = src.shape[0]
    nw = _NC * _NS
    e_per = e // nw
    rowb = 80
    nb = _cdiv(n, rowb)
    assert e % nw == 0 and e_per % k_chunk == 0 and n % rowb == 0
    n_chunks = e_per // k_chunk
    mesh = plsc.VectorSubcoreMesh(core_axis_name="c", subcore_axis_name="s")

    @functools.partial(
        pl.kernel,
        out_type=jax.ShapeDtypeStruct((_NC, n, d), jnp.float32),
        mesh=mesh,
        scratch_types=[
            pltpu.VMEM_SHARED((n, d), jnp.float32),
            pltpu.VMEM((k_chunk,), jnp.int32),
            pltpu.VMEM((k_chunk,), jnp.int32),
            pltpu.VMEM((k_chunk, d), jnp.float32),
            pltpu.SemaphoreType.DMA,
        ],
    )
    def k(h_hbm, src_hbm, dst_hbm, out_hbm, acc, sidx, didx, rows, sem):
        cid = lax.axis_index("c")
        sid = lax.axis_index("s")
        for jj in range(_cdiv(nb, _NS)):
            j = jj * _NS + sid

            @pl.when(j < nb)
            def _():
                pltpu.sync_copy(h_hbm.at[pl.ds(j * rowb, rowb)],
                                acc.at[pl.ds(j * rowb, rowb)])
        plsc.subcore_barrier()
        wid = cid * _NS + sid

        def body(j, carry):
            base = wid * e_per + j * k_chunk
            ci = pltpu.async_copy(src_hbm.at[pl.ds(base, k_chunk)], sidx, sem)
            cd = pltpu.async_copy(dst_hbm.at[pl.ds(base, k_chunk)], didx, sem)
            ci.wait()
            cd.wait()
            pltpu.async_copy(h_hbm.at[sidx], rows, sem).wait()
            pltpu.sync_copy(rows, acc.at[didx], add=True)
            return carry

        lax.fori_loop(0, n_chunks, body, 0)
        plsc.subcore_barrier()
        for jj in range(_cdiv(nb, _NS)):
            j = jj * _NS + sid

            @pl.when(j < nb)
            def _():
                pltpu.sync_copy(acc.at[pl.ds(j * rowb, rowb)],
                                out_hbm.at[cid, pl.ds(j * rowb, rowb)])

    return k(h, src, dst)


def _mm_comb_body(p0_ref, p1_ref, h_ref, w_ref, b_ref, o_ref):
    x = p0_ref[0] + p1_ref[0] - h_ref[...]
    r = jnp.dot(x, w_ref[...], preferred_element_type=jnp.float32)
    o_ref[...] = jnp.maximum(r + b_ref[...], 0.0)


def _mm_comb(p, h, w, b, bm=1024):
    n, d = h.shape
    _, nn = w.shape
    grid = (_cdiv(n, bm),)
    return pl.pallas_call(
        _mm_comb_body,
        grid=grid,
        in_specs=[
            pl.BlockSpec((1, bm, d), lambda i: (0, i, 0)),
            pl.BlockSpec((1, bm, d), lambda i: (1, i, 0)),
            pl.BlockSpec((bm, d), lambda i: (i, 0)),
            pl.BlockSpec((d, nn), lambda i: (0, 0)),
            pl.BlockSpec((1, nn), lambda i: (0, 0)),
        ],
        out_specs=pl.BlockSpec((bm, nn), lambda i: (i, 0)),
        out_shape=jax.ShapeDtypeStruct((n, nn), jnp.float32),
        compiler_params=pltpu.CompilerParams(
            dimension_semantics=("parallel",)),
    )(p, p, h, w, b.reshape(1, -1))


def _sc_gat(tab, ed_col, src, dst, k_chunk=64):
    n = tab.shape[0]
    e = src.shape[0]
    nw = _NC * _NS
    e_per = e // nw
    assert e % nw == 0 and e_per % k_chunk == 0
    n_chunks = e_per // k_chunk
    rowb = 128
    nb = _cdiv(n, rowb)
    ngr = k_chunk // 16
    mesh = plsc.VectorSubcoreMesh(core_axis_name="c", subcore_axis_name="s")

    @functools.partial(
        pl.kernel,
        out_type=(jax.ShapeDtypeStruct((_NC, n, 16), jnp.float32),
                  jax.ShapeDtypeStruct((_NC, n), jnp.float32)),
        mesh=mesh,
        scratch_types=[
            pltpu.VMEM_SHARED((n, 16), jnp.float32),
            pltpu.VMEM_SHARED((n,), jnp.float32),
            pltpu.VMEM((k_chunk,), jnp.float32),
            pltpu.VMEM((k_chunk,), jnp.int32),
            pltpu.VMEM((k_chunk,), jnp.int32),
            pltpu.VMEM((k_chunk, 18), jnp.float32),
            pltpu.VMEM((k_chunk, 16), jnp.float32),
            pltpu.VMEM((k_chunk,), jnp.float32),
            pltpu.VMEM((rowb, 16), jnp.float32),
            pltpu.VMEM((rowb,), jnp.float32),
            pltpu.SemaphoreType.DMA,
        ],
        compiler_params=pltpu.CompilerParams(
            needs_layout_passes=False, use_tc_tiling_on_sc=False),
    )
    def k(tab_hbm, ed_hbm, src_hbm, dst_hbm,
          on_hbm, od_hbm, accn, accd, edb, sidx, didx, rows, pay, exb,
          zb, zdb, sem):
        cid = lax.axis_index("c")
        sid = lax.axis_index("s")
        wid = cid * _NS + sid
        z16 = jnp.zeros((16,), jnp.float32)
        for r in range(rowb):
            zb[r, :] = z16
        for r in range(rowb // 16):
            zdb[pl.ds(r * 16, 16)] = z16
        for jj in range(_cdiv(nb, _NS)):
            j = jj * _NS + sid

            @pl.when(j < nb)
            def _():
                pltpu.sync_copy(zb, accn.at[pl.ds(j * rowb, rowb)])
                pltpu.sync_copy(zdb, accd.at[pl.ds(j * rowb, rowb)])
        plsc.subcore_barrier()

        iota = lax.iota(jnp.int32, 16)
        col16 = jnp.full((16,), 16, jnp.int32)

        def body(j, carry):
            base = wid * e_per + j * k_chunk
            ci = pltpu.async_copy(src_hbm.at[pl.ds(base, k_chunk)], sidx, sem)
            cd = pltpu.async_copy(dst_hbm.at[pl.ds(base, k_chunk)], didx, sem)
            ci.wait()
            cd.wait()
            cp1 = pltpu.async_copy(tab_hbm.at[sidx], rows, sem)
            cp2 = pltpu.async_copy(ed_hbm.at[didx], edb, sem)
            cp1.wait()
            cp2.wait()
            for g in range(ngr):
                ed_vec = edb[pl.ds(g * 16, 16)]
                es_vec = plsc.load_gather(rows, [iota + g * 16, col16])
                ee = es_vec + ed_vec
                ee = jnp.where(ee >= 0, ee, 0.2 * ee)
                ex = jnp.exp(ee)
                exb[pl.ds(g * 16, 16)] = ex
                for lane in range(16):
                    j2 = g * 16 + lane
                    pay[j2, :] = rows[j2, pl.ds(0, 16)] * ex[lane]
            pltpu.sync_copy(pay, accn.at[didx], add=True)
            pltpu.sync_copy(exb, accd.at[didx], add=True)
            return carry

        lax.fori_loop(0, n_chunks, body, 0)
        plsc.subcore_barrier()
        for jj in range(_cdiv(nb, _NS)):
            j = jj * _NS + sid

            @pl.when(j < nb)
            def _():
                pltpu.sync_copy(accn.at[pl.ds(j * rowb, rowb)],
                                on_hbm.at[cid, pl.ds(j * rowb, rowb)])
                pltpu.sync_copy(accd.at[pl.ds(j * rowb, rowb)],
                                od_hbm.at[cid, pl.ds(j * rowb, rowb)])

    return k(tab, ed_col, src, dst)


def _gat_finish_body(h_ref, pn_ref, pd_ref, o_ref):
    hb = h_ref[...]
    h16 = hb[:, 0:16]
    ee = hb[:, 16:17] + hb[:, 17:18]
    ee = jnp.where(ee >= 0, ee, 0.2 * ee)
    w = jnp.exp(ee)
    num = pn_ref[0] + pn_ref[1] + w * h16
    den = pd_ref[0] + pd_ref[1] + w
    o_ref[...] = jnp.maximum(num / den, 0.0)


def _gat_finish(h_aug, pn, pd, bm=2048):
    n = h_aug.shape[0]
    pd3 = pd.reshape(_NC, n, 1)
    grid = (_cdiv(n, bm),)
    return pl.pallas_call(
        _gat_finish_body,
        grid=grid,
        in_specs=[
            pl.BlockSpec((bm, 18), lambda i: (i, 0)),
            pl.BlockSpec((_NC, bm, 16), lambda i: (0, i, 0)),
            pl.BlockSpec((_NC, bm, 1), lambda i: (0, i, 0)),
        ],
        out_specs=pl.BlockSpec((bm, 16), lambda i: (i, 0)),
        out_shape=jax.ShapeDtypeStruct((n, 16), jnp.float32),
        compiler_params=pltpu.CompilerParams(
            dimension_semantics=("parallel",)),
    )(h_aug, pn, pd3)


def _sc_pool(x, fidx, lidx, k_chunk=64):
    n_out = fidx.shape[0]
    nw = _NC * _NS
    per = n_out // nw
    assert n_out % nw == 0 and per % k_chunk == 0
    n_chunks = per // k_chunk
    mesh = plsc.VectorSubcoreMesh(core_axis_name="c", subcore_axis_name="s")

    @functools.partial(
        pl.kernel,
        out_type=jax.ShapeDtypeStruct((n_out, 16), jnp.float32),
        mesh=mesh,
        scratch_types=[
            pltpu.VMEM((k_chunk,), jnp.int32),
            pltpu.VMEM((k_chunk,), jnp.int32),
            pltpu.VMEM((k_chunk, 16), jnp.float32),
            pltpu.VMEM((k_chunk, 16), jnp.float32),
            pltpu.VMEM((k_chunk, 16), jnp.float32),
            pltpu.SemaphoreType.DMA,
        ],
        compiler_params=pltpu.CompilerParams(
            needs_layout_passes=False, use_tc_tiling_on_sc=False),
    )
    def k(x_hbm, f_hbm, l_hbm, out_hbm, fi, li, ra, rb, rc, sem):
        cid = lax.axis_index("c")
        sid = lax.axis_index("s")
        wid = cid * _NS + sid

        def body(j, carry):
            base = wid * per + j * k_chunk
            pltpu.sync_copy(f_hbm.at[pl.ds(base, k_chunk)], fi)
            pltpu.sync_copy(l_hbm.at[pl.ds(base, k_chunk)], li)
            pltpu.async_copy(x_hbm.at[fi], ra, sem).wait()
            pltpu.async_copy(x_hbm.at[li], rb, sem).wait()
            for j2 in range(k_chunk):
                rc[j2, :] = jnp.maximum(ra[j2, pl.ds(0, 16)],
                                        rb[j2, pl.ds(0, 16)])
            pltpu.sync_copy(rc, out_hbm.at[pl.ds(base, k_chunk)])
            return carry

        lax.fori_loop(0, n_chunks, body, 0)

    return k(x, fidx, lidx)


def _sc_segmax(x, starts, segs_per_tile, max_len):
    n, d = x.shape
    n_seg = starts.shape[0] - 1
    nw = _NC * _NS
    assert n_seg == nw * segs_per_tile
    nst = starts.shape[0]
    nst_pad = _cdiv(nst, 8) * 8
    starts_pad = jnp.concatenate(
        [starts, jnp.zeros((nst_pad - nst,), jnp.int32)])
    mesh = plsc.VectorSubcoreMesh(core_axis_name="c", subcore_axis_name="s")
    ml = _cdiv(max_len, 16) * 16

    @functools.partial(
        pl.kernel,
        out_type=jax.ShapeDtypeStruct((nw, segs_per_tile, d), jnp.float32),
        mesh=mesh,
        scratch_types=[
            pltpu.VMEM((nst_pad,), jnp.int32),
            pltpu.VMEM((ml,), jnp.int32),
            pltpu.VMEM((ml, d), jnp.float32),
            pltpu.VMEM((segs_per_tile, d), jnp.float32),
            pltpu.SemaphoreType.DMA,
        ],
        compiler_params=pltpu.CompilerParams(
            needs_layout_passes=False, use_tc_tiling_on_sc=False),
    )
    def k(x_hbm, st_hbm, out_hbm, stv, idxb, buf, ob, sem):
        cid = lax.axis_index("c")
        sid = lax.axis_index("s")
        wid = cid * _NS + sid
        pltpu.sync_copy(st_hbm, stv)
        iota = lax.iota(jnp.int32, 16)
        for t in range(segs_per_tile):
            seg = wid * segs_per_tile + t
            sv = plsc.load_gather(stv, [jnp.minimum(iota + seg, nst - 1)])
            s0 = sv[0]
            s1 = sv[1]
            for i in range(ml // 16):
                idxb[pl.ds(i * 16, 16)] = jnp.minimum(
                    iota + s0 + i * 16, n - 1)
            pltpu.async_copy(x_hbm.at[idxb], buf, sem).wait()
            for c in range(d // 16):
                acc = buf[0, pl.ds(c * 16, 16)]

                def red(r, a):
                    return jnp.maximum(a, buf[r, pl.ds(c * 16, 16)])

                acc = lax.fori_loop(1, s1 - s0, red, acc)
                ob[t, pl.ds(c * 16, 16)] = acc
        pltpu.sync_copy(ob, out_hbm.at[wid])

    return k(x, starts_pad).reshape(n_seg, d)


def kernel(drug_x, cell_x, W1, b1, W2, b2, W3, b3, Wd, bd, gW0, gas0, gad0,
           gW1, gas1, gad1, Wc1, bc1, Wc2, bc2, Wr1, br1, Wr2, br2, Wr3, br3,
           drug_edge_index, drug_batch, cell_edge_index, cluster0, cluster1):
    n_drug = drug_x.shape[0]
    n_cell = cell_x.shape[0]
    b = B_K
    genes = cluster0.shape[0]
    c0 = cluster1.shape[0]
    c1 = C1_K

    src, dst = drug_edge_index[0], drug_edge_index[1]

    h = drug_x
    src32 = src.astype(jnp.int32)
    dst32 = dst.astype(jnp.int32)
    reps = []
    for (W, bb) in ((W1, b1), (W2, b2), (W3, b3)):
        p = _sc_segsum(h, src32, dst32)
        h = _mm_comb(p, h, W, bb)
        reps.append(h)
    node_rep = jnp.concatenate(reps, axis=-1)
    starts = jnp.searchsorted(drug_batch.astype(jnp.int32),
                              jnp.arange(b + 1, dtype=jnp.int32)
                              ).astype(jnp.int32)
    x_drug = _sc_segmax(node_rep, starts, b // (_NC * _NS), 80)
    x_drug = _mm(x_drug, Wd, bd, act=1)

    csrc = cell_edge_index[0].astype(jnp.int32)
    cdst = cell_edge_index[1].astype(jnp.int32)
    va_s0 = gW0 @ gas0
    va_d0 = gW0 @ gad0
    W0_aug = jnp.concatenate([gW0, va_s0[:, None], va_d0[:, None]], axis=1)
    h_aug0 = _mm(cell_x, W0_aug, jnp.zeros((18,), jnp.float32))
    pn0, pd0 = _sc_gat(h_aug0, h_aug0[:, 17], csrc, cdst)
    h1 = _gat_finish(h_aug0, pn0, pd0)

    c0_t = cluster0.astype(jnp.int32)
    c1_t = cluster1.astype(jnp.int32)
    n1 = b * c0
    ar = jnp.arange(c0, dtype=jnp.int32)
    first0 = jnp.searchsorted(c0_t, ar).astype(jnp.int32)
    last0 = (jnp.searchsorted(c0_t, ar, side='right') - 1).astype(jnp.int32)
    boff = (jnp.arange(b, dtype=jnp.int32) * genes)[:, None]
    fidx0 = (boff + first0[None, :]).reshape(-1)
    lidx0 = (boff + last0[None, :]).reshape(-1)
    x = _sc_pool(h1, fidx0, lidx0)
    csrc2 = c0_t[csrc % genes] + (csrc // genes) * c0
    cdst2 = c0_t[cdst % genes] + (cdst // genes) * c0
    x = _bn_pl(x)

    va_s1 = gW1 @ gas1
    va_d1 = gW1 @ gad1
    W1_aug = jnp.concatenate([gW1, va_s1[:, None], va_d1[:, None]], axis=1)
    h_aug1 = _mm(x, W1_aug, jnp.zeros((18,), jnp.float32))
    pn1, pd1 = _sc_gat(h_aug1, h_aug1[:, 17], csrc2, cdst2)
    h2 = _gat_finish(h_aug1, pn1, pd1)

    n2 = b * c1
    ar1 = jnp.arange(c1, dtype=jnp.int32)
    first1 = jnp.searchsorted(c1_t, ar1).astype(jnp.int32)
    last1 = (jnp.searchsorted(c1_t, ar1, side='right') - 1).astype(jnp.int32)
    boff1 = (jnp.arange(b, dtype=jnp.int32) * c0)[:, None]
    fidx1 = (boff1 + first1[None, :]).reshape(-1)
    lidx1 = (boff1 + last1[None, :]).reshape(-1)
    x = _sc_pool(h2, fidx1, lidx1)
    x = _bn_pl(x)

    x_cell = x.reshape(b, c1 * 16)
    x_cell = _mm(x_cell, Wc1, bc1, act=1, bm=b)
    x_cell = _mm(x_cell, Wc2, bc2, act=1, bm=b)

    z = jnp.concatenate([x_drug, x_cell], axis=-1)
    z = _mm(z, Wr1, br1, act=2, bm=b)
    z = _mm(z, Wr2, br2, act=2, bm=b)
    w3p = jnp.concatenate([Wr3, jnp.zeros((Wr3.shape[0], 127), jnp.float32)],
                          axis=1)
    b3p = jnp.concatenate([br3, jnp.zeros((127,), jnp.float32)])
    out = _mm(z, w3p, b3p, act=0, bm=b)
    return out[:, :1]
